# baseline (speedup 1.0000x reference)
import functools

import jax
import jax.numpy as jnp
from jax import lax
from jax.experimental import pallas as pl
from jax.experimental.pallas import tpu as pltpu
from jax.experimental.pallas import tpu_sc as plsc

_NC, _NS = 2, 16
_NW = _NC * _NS
_E = 320000
_BPW = _E // _NW
_CH = 80
_NCHUNK = _BPW // _CH


def _sc_gather2_body(ti_hbm, tj_hbm, dst_hbm, src_hbm, g1_hbm, g2_hbm,
                     idx_d_v, idx_s_v, rows1_v, rows2_v, sem1, sem2):
    wid = lax.axis_index("s") * _NC + lax.axis_index("c")
    base0 = wid * _BPW
    pltpu.sync_copy(dst_hbm.at[wid], idx_d_v)
    pltpu.sync_copy(src_hbm.at[wid], idx_s_v)

    def body(i, carry):
        base = base0 + i * _CH
        cp1 = pltpu.async_copy(ti_hbm.at[idx_d_v.at[i]], rows1_v, sem1)
        cp2 = pltpu.async_copy(tj_hbm.at[idx_s_v.at[i]], rows2_v, sem2)
        cp1.wait()
        cp2.wait()
        pltpu.sync_copy(rows1_v, g1_hbm.at[pl.ds(base, _CH)])
        pltpu.sync_copy(rows2_v, g2_hbm.at[pl.ds(base, _CH)])
        return carry

    lax.fori_loop(0, _NCHUNK, body, 0)


_NP = 10240
_TSL = _NP // _NS


def _sc_denom_body(logits_hbm, dst_hbm, gmax_hbm, ex_hbm, denom_hbm,
                   logit_v, ex_v, idx_v, gmax_v, zbuf_v, den_sh):
    sid = lax.axis_index("s")
    cid = lax.axis_index("c")
    wid = sid * _NC + cid
    z16 = jnp.zeros((16,), jnp.float32)
    for g in range(_TSL // 16):
        zbuf_v[pl.ds(g * 16, 16)] = z16
    pltpu.sync_copy(zbuf_v, den_sh.at[pl.ds(sid * _TSL, _TSL)])
    pltpu.sync_copy(logits_hbm.at[wid], logit_v)
    pltpu.sync_copy(dst_hbm.at[wid], idx_v)
    pltpu.sync_copy(gmax_hbm, gmax_v)
    gmax = gmax_v[...]
    plsc.subcore_barrier()

    def body(i, carry):
        for g in range(_CH // 16):
            lv = logit_v[i, pl.ds(g * 16, 16)]
            ex_v[i, pl.ds(g * 16, 16)] = jnp.exp(lv - gmax)
        pltpu.sync_copy(ex_v.at[i], den_sh.at[idx_v.at[i]], add=True)
        return carry

    lax.fori_loop(0, _NCHUNK, body, 0)
    pltpu.sync_copy(ex_v, ex_hbm.at[wid])
    plsc.subcore_barrier()
    pltpu.sync_copy(den_sh.at[pl.ds(sid * _TSL, _TSL)],
                    denom_hbm.at[cid, pl.ds(sid * _TSL, _TSL)])


def _sc_denom(logits3, dst3, gmax16):
    mesh = plsc.VectorSubcoreMesh(core_axis_name="c", subcore_axis_name="s")
    f = pl.kernel(
        _sc_denom_body,
        mesh=mesh,
        out_type=[
            jax.ShapeDtypeStruct((_NW, _NCHUNK, _CH), jnp.float32),
            jax.ShapeDtypeStruct((_NC, _NP), jnp.float32),
        ],
        scratch_types=[
            pltpu.VMEM((_NCHUNK, _CH), jnp.float32),
            pltpu.VMEM((_NCHUNK, _CH), jnp.float32),
            pltpu.VMEM((_NCHUNK, _CH), jnp.int32),
            pltpu.VMEM((16,), jnp.float32),
            pltpu.VMEM((_TSL,), jnp.float32),
            pltpu.VMEM_SHARED((_NP,), jnp.float32),
        ],
    )
    return f(logits3, dst3, gmax16)


_NP2 = _NP // 2
_TSL2 = _NP2 // _NS
_NCHUNK2 = _E // _NS // _CH


_NPQ = _NP // 4
_TSLQ = _NPQ // _NS


def _sc_msg_body(tn_hbm, src_hbm, dst_hbm, ex_hbm, denom_hbm, out_hbm,
                 idx_s_v, idx_d_v, idx_c_v, ex_v, alpha_v, d_v, rows_v,
                 acc_sh, sem, sem2):
    sid = lax.axis_index("s")
    cid = lax.axis_index("c")
    z16 = jnp.zeros((16,), jnp.float32)
    z16i = jnp.zeros((16,), jnp.int32)
    zf16 = jnp.zeros((16,), jnp.float32)
    pltpu.sync_copy(src_hbm.at[sid], idx_s_v)
    pltpu.sync_copy(dst_hbm.at[sid], idx_d_v)
    pltpu.sync_copy(ex_hbm.at[sid], ex_v)

    for p in range(2):
        for r in range(_CH):
            for g in range(8):
                rows_v[r, pl.ds(g * 16, 16)] = z16
        for b in range(_TSLQ // _CH):
            pltpu.sync_copy(rows_v, acc_sh.at[pl.ds(sid * _TSLQ + b * _CH, _CH)])
        plsc.subcore_barrier()
        lo = cid * _NP2 + p * _NPQ

        def body(i, carry):
            cp1 = pltpu.async_copy(tn_hbm.at[idx_s_v.at[i]], rows_v, sem)
            cp2 = pltpu.async_copy(denom_hbm.at[idx_d_v.at[i]], d_v, sem2)
            cp1.wait()
            cp2.wait()
            for g in range(_CH // 16):
                d16 = d_v[pl.ds(g * 16, 16)]
                ex16 = ex_v[i, pl.ds(g * 16, 16)]
                dv = idx_d_v[i, pl.ds(g * 16, 16)] - lo
                inb = (dv >= 0) & (dv < _NPQ)
                idx_c_v[pl.ds(g * 16, 16)] = jnp.where(inb, dv, z16i)
                alpha_v[pl.ds(g * 16, 16)] = jnp.where(inb, ex16 / d16, zf16)
            for g16 in range(_CH // 16):
                av16 = alpha_v[pl.ds(g16 * 16, 16)]
                for j in range(16):
                    r = g16 * 16 + j
                    ar = av16[j]
                    for g in range(8):
                        rows_v[r, pl.ds(g * 16, 16)] = (
                            rows_v[r, pl.ds(g * 16, 16)] * ar)
            pltpu.sync_copy(rows_v, acc_sh.at[idx_c_v], add=True)
            return carry

        lax.fori_loop(0, _NCHUNK2, body, 0)
        plsc.subcore_barrier()
        pltpu.sync_copy(acc_sh.at[pl.ds(sid * _TSLQ, _TSLQ)],
                        out_hbm.at[cid * 2 + p, pl.ds(sid * _TSLQ, _TSLQ)])
        plsc.subcore_barrier()


def _sc_msg(table_n, src2, dst2, ex2, denom):
    h = table_n.shape[1]
    mesh = plsc.VectorSubcoreMesh(core_axis_name="c", subcore_axis_name="s")
    f = pl.kernel(
        _sc_msg_body,
        mesh=mesh,
        out_type=jax.ShapeDtypeStruct((4, _NPQ, h), jnp.float32),
        scratch_types=[
            pltpu.VMEM((_NCHUNK2, _CH), jnp.int32),
            pltpu.VMEM((_NCHUNK2, _CH), jnp.int32),
            pltpu.VMEM((_CH,), jnp.int32),
            pltpu.VMEM((_NCHUNK2, _CH), jnp.float32),
            pltpu.VMEM((_CH,), jnp.float32),
            pltpu.VMEM((_CH,), jnp.float32),
            pltpu.VMEM((_CH, h), jnp.float32),
            pltpu.VMEM_SHARED((_NPQ, h), jnp.float32),
            pltpu.SemaphoreType.DMA,
            pltpu.SemaphoreType.DMA,
        ],
    )
    return f(table_n, src2, dst2, ex2, denom)


def _sc_gather2(table_i, table_j, dst, src):
    h = table_i.shape[1]
    dst3 = dst.reshape(_NW, _NCHUNK, _CH)
    src3 = src.reshape(_NW, _NCHUNK, _CH)
    mesh = plsc.VectorSubcoreMesh(core_axis_name="c", subcore_axis_name="s")
    f = pl.kernel(
        _sc_gather2_body,
        mesh=mesh,
        out_type=[
            jax.ShapeDtypeStruct((_E, h), jnp.float32),
            jax.ShapeDtypeStruct((_E, h), jnp.float32),
        ],
        scratch_types=[
            pltpu.VMEM((_NCHUNK, _CH), jnp.int32),
            pltpu.VMEM((_NCHUNK, _CH), jnp.int32),
            pltpu.VMEM((_CH, h), jnp.float32),
            pltpu.VMEM((_CH, h), jnp.float32),
            pltpu.SemaphoreType.DMA,
            pltpu.SemaphoreType.DMA,
        ],
    )
    return f(table_i, table_j, dst3, src3)


_PREC = jax.lax.Precision.HIGHEST


def _tc_proj_kernel(relu, x_ref, w_ref, b_ref, out_ref):
    x = x_ref[...]
    if relu:
        x = jnp.maximum(x, 0.0)
    out_ref[...] = (jnp.dot(x, w_ref[...], precision=_PREC,
                            preferred_element_type=jnp.float32) + b_ref[0])


def _tc_proj(x, w, b, relu):
    n, k = x.shape
    m = w.shape[1]
    blk = 2000
    return pl.pallas_call(
        functools.partial(_tc_proj_kernel, relu),
        grid=(n // blk,),
        in_specs=[
            pl.BlockSpec((blk, k), lambda i: (i, 0)),
            pl.BlockSpec((k, m), lambda i: (0, 0)),
            pl.BlockSpec(memory_space=pltpu.SMEM),
        ],
        out_specs=pl.BlockSpec((blk, m), lambda i: (i, 0)),
        out_shape=jax.ShapeDtypeStruct((n, m), jnp.float32),
    )(x, w, b)


def _tc_fused(g1, g2, c, we_c, av, we_next):
    e, h = g1.shape
    kc = c.shape[1]
    blk = 2000
    has_wec = we_c is not None
    has_next = we_next is not None
    av2 = av.reshape(h, 1)

    def kern(*refs):
        it = iter(refs)
        g1_ref = next(it)
        g2_ref = next(it)
        c_ref = next(it)
        we_ref = next(it) if has_wec else None
        av_ref = next(it)
        wn_ref = next(it) if has_next else None
        logit_ref = next(it)
        gmax_ref = next(it)
        cn_ref = next(it) if has_next else None
        i = pl.program_id(0)
        if has_wec:
            cterm = jnp.dot(c_ref[...], we_ref[...], precision=_PREC,
                            preferred_element_type=jnp.float32)
        else:
            cterm = c_ref[...]
        f = g1_ref[...] + g2_ref[...] + cterm
        e_act = jnp.where(f > 0, f, 0.2 * f)
        logits = jnp.dot(e_act, av_ref[...], precision=_PREC,
                         preferred_element_type=jnp.float32)
        logit_ref[...] = logits
        bmax = jnp.max(logits)

        @pl.when(i == 0)
        def _():
            gmax_ref[0, 0] = bmax

        @pl.when(i > 0)
        def _():
            gmax_ref[0, 0] = jnp.maximum(gmax_ref[0, 0], bmax)

        if has_next:
            cn_ref[...] = jnp.dot(f, wn_ref[...], precision=_PREC,
                                  preferred_element_type=jnp.float32)

    in_specs = [
        pl.BlockSpec((blk, h), lambda i: (i, 0)),
        pl.BlockSpec((blk, h), lambda i: (i, 0)),
        pl.BlockSpec((blk, kc), lambda i: (i, 0)),
    ]
    args = [g1, g2, c]
    if has_wec:
        in_specs.append(pl.BlockSpec((kc, h), lambda i: (0, 0)))
        args.append(we_c)
    in_specs.append(pl.BlockSpec((h, 1), lambda i: (0, 0)))
    args.append(av2)
    out_specs = [
        pl.BlockSpec((blk, 1), lambda i: (i, 0)),
        pl.BlockSpec((1, 1), lambda i: (0, 0), memory_space=pltpu.SMEM),
    ]
    out_shape = [
        jax.ShapeDtypeStruct((e, 1), jnp.float32),
        jax.ShapeDtypeStruct((1, 1), jnp.float32),
    ]
    if has_next:
        in_specs.append(pl.BlockSpec((h, h), lambda i: (0, 0)))
        args.append(we_next)
        out_specs.append(pl.BlockSpec((blk, h), lambda i: (i, 0)))
        out_shape.append(jax.ShapeDtypeStruct((e, h), jnp.float32))
    res = pl.pallas_call(
        kern, grid=(e // blk,), in_specs=in_specs,
        out_specs=out_specs, out_shape=out_shape,
    )(*args)
    c_next = res[2] if has_next else None
    return res[0].reshape(e), res[1][0, 0], c_next


def _layer(x, src, dst, c, we_c, Wn, Wi, Wj, av, n, We_next, relu_in):
    wcat = jnp.concatenate([Wi, Wj, Wn], axis=1)
    zb = jnp.zeros((1,), jnp.float32)
    tbl = _tc_proj(x, wcat, zb, relu_in)
    h = Wn.shape[1]
    xWi = tbl[:, :h]
    xWj = tbl[:, h:2 * h]
    xWn = tbl[:, 2 * h:]
    g1, g2 = _sc_gather2(xWi, xWj, dst, src)
    logits, gmax, c_next = _tc_fused(g1, g2, c, we_c, av, We_next)
    logits3 = logits.reshape(_NW, _NCHUNK, _CH)
    dst3 = dst.reshape(_NW, _NCHUNK, _CH)
    gmax16 = jnp.full((16,), gmax, jnp.float32)
    ex3, denom2 = _sc_denom(logits3, dst3, gmax16)
    denom = denom2[0] + denom2[1] + 1e-16
    src2 = src.reshape(_NS, _NCHUNK2, _CH)
    dst2 = dst.reshape(_NS, _NCHUNK2, _CH)
    ex2 = ex3.reshape(_NS, _NCHUNK2, _CH)
    out4 = _sc_msg(xWn, src2, dst2, ex2, denom)
    out = out4.reshape(4 * _NPQ, -1)[:n]
    return out, c_next


def kernel(x, edge_index, edge_attr, Wn1, Wi1, Wj1, We1, av1, Wn2, Wi2, Wj2, We2, av2, Wn3, Wi3, Wj3, We3, av3, Wc, bc):
    n = x.shape[0]
    src = edge_index[0]
    dst = edge_index[1]
    h, c2 = _layer(x, src, dst, edge_attr, We1, Wn1, Wi1, Wj1, av1, n, We2,
                   relu_in=False)
    h, c3 = _layer(h, src, dst, c2, None, Wn2, Wi2, Wj2, av2, n, We3,
                   relu_in=True)
    h, _ = _layer(h, src, dst, c3, None, Wn3, Wi3, Wj3, av3, n, None,
                  relu_in=True)
    return _tc_proj(h, Wc, bc, relu=True)


# speedup vs baseline: 1.0908x; 1.0908x over previous
import functools

import jax
import jax.numpy as jnp
from jax import lax
from jax.experimental import pallas as pl
from jax.experimental.pallas import tpu as pltpu
from jax.experimental.pallas import tpu_sc as plsc

_NC, _NS = 2, 16
_NW = _NC * _NS
_E = 320000
_BPW = _E // _NW
_CH = 80
_NCHUNK = _BPW // _CH


def _sc_gather2_body(ti_hbm, tj_hbm, dst_hbm, src_hbm, g1_hbm, g2_hbm,
                     idx_d_v, idx_s_v, rows1_v, rows2_v, sem1, sem2):
    wid = lax.axis_index("s") * _NC + lax.axis_index("c")
    base0 = wid * _BPW
    pltpu.sync_copy(dst_hbm.at[wid], idx_d_v)
    pltpu.sync_copy(src_hbm.at[wid], idx_s_v)

    def body(i, carry):
        base = base0 + i * _CH
        cp1 = pltpu.async_copy(ti_hbm.at[idx_d_v.at[i]], rows1_v, sem1)
        cp2 = pltpu.async_copy(tj_hbm.at[idx_s_v.at[i]], rows2_v, sem2)
        cp1.wait()
        cp2.wait()
        pltpu.sync_copy(rows1_v, g1_hbm.at[pl.ds(base, _CH)])
        pltpu.sync_copy(rows2_v, g2_hbm.at[pl.ds(base, _CH)])
        return carry

    lax.fori_loop(0, _NCHUNK, body, 0)


_NP = 10240
_TSL = _NP // _NS


def _sc_denom_body(logits_hbm, dst_hbm, gmax_hbm, ex_hbm, denom_hbm,
                   logit_v, ex_v, idx_v, gmax_v, zbuf_v, den_sh):
    sid = lax.axis_index("s")
    cid = lax.axis_index("c")
    wid = sid * _NC + cid
    z16 = jnp.zeros((16,), jnp.float32)
    for g in range(_TSL // 16):
        zbuf_v[pl.ds(g * 16, 16)] = z16
    pltpu.sync_copy(zbuf_v, den_sh.at[pl.ds(sid * _TSL, _TSL)])
    pltpu.sync_copy(logits_hbm.at[wid], logit_v)
    pltpu.sync_copy(dst_hbm.at[wid], idx_v)
    pltpu.sync_copy(gmax_hbm, gmax_v)
    gmax = gmax_v[...]
    plsc.subcore_barrier()

    def body(i, carry):
        for g in range(_CH // 16):
            lv = logit_v[i, pl.ds(g * 16, 16)]
            ex_v[i, pl.ds(g * 16, 16)] = jnp.exp(lv - gmax)
        pltpu.sync_copy(ex_v.at[i], den_sh.at[idx_v.at[i]], add=True)
        return carry

    lax.fori_loop(0, _NCHUNK, body, 0)
    pltpu.sync_copy(ex_v, ex_hbm.at[wid])
    plsc.subcore_barrier()
    pltpu.sync_copy(den_sh.at[pl.ds(sid * _TSL, _TSL)],
                    denom_hbm.at[cid, pl.ds(sid * _TSL, _TSL)])


def _sc_denom(logits3, dst3, gmax16):
    mesh = plsc.VectorSubcoreMesh(core_axis_name="c", subcore_axis_name="s")
    f = pl.kernel(
        _sc_denom_body,
        mesh=mesh,
        out_type=[
            jax.ShapeDtypeStruct((_NW, _NCHUNK, _CH), jnp.float32),
            jax.ShapeDtypeStruct((_NC, _NP), jnp.float32),
        ],
        scratch_types=[
            pltpu.VMEM((_NCHUNK, _CH), jnp.float32),
            pltpu.VMEM((_NCHUNK, _CH), jnp.float32),
            pltpu.VMEM((_NCHUNK, _CH), jnp.int32),
            pltpu.VMEM((16,), jnp.float32),
            pltpu.VMEM((_TSL,), jnp.float32),
            pltpu.VMEM_SHARED((_NP,), jnp.float32),
        ],
    )
    return f(logits3, dst3, gmax16)


_NP2 = _NP // 2
_TSL2 = _NP2 // _NS
_NCHUNK2 = _E // _NS // _CH


_NPQ = _NP // 4
_TSLQ = _NPQ // _NS


def _sc_msg_body(tn_hbm, src_hbm, dst_hbm, ex_hbm, denom_hbm, out_hbm,
                 idx_s_v, idx_d_v, idx_c_v, ex_v, alpha_v, d_v, rows_v,
                 acc_sh, sem, sem2):
    sid = lax.axis_index("s")
    cid = lax.axis_index("c")
    z16 = jnp.zeros((16,), jnp.float32)
    z16i = jnp.zeros((16,), jnp.int32)
    zf16 = jnp.zeros((16,), jnp.float32)
    pltpu.sync_copy(src_hbm.at[sid], idx_s_v)
    pltpu.sync_copy(dst_hbm.at[sid], idx_d_v)
    pltpu.sync_copy(ex_hbm.at[sid], ex_v)

    for p in range(2):
        for r in range(_CH):
            for g in range(8):
                rows_v[r, pl.ds(g * 16, 16)] = z16
        for b in range(_TSLQ // _CH):
            pltpu.sync_copy(rows_v, acc_sh.at[pl.ds(sid * _TSLQ + b * _CH, _CH)])
        plsc.subcore_barrier()
        lo = cid * _NP2 + p * _NPQ

        def body(i, carry):
            cp1 = pltpu.async_copy(tn_hbm.at[idx_s_v.at[i]], rows_v, sem)
            cp2 = pltpu.async_copy(denom_hbm.at[idx_d_v.at[i]], d_v, sem2)
            cp1.wait()
            cp2.wait()
            for g in range(_CH // 16):
                d16 = d_v[pl.ds(g * 16, 16)]
                ex16 = ex_v[i, pl.ds(g * 16, 16)]
                dv = idx_d_v[i, pl.ds(g * 16, 16)] - lo
                inb = (dv >= 0) & (dv < _NPQ)
                idx_c_v[pl.ds(g * 16, 16)] = jnp.where(inb, dv, z16i)
                alpha_v[pl.ds(g * 16, 16)] = jnp.where(inb, ex16 / d16, zf16)
            for g16 in range(_CH // 16):
                av16 = alpha_v[pl.ds(g16 * 16, 16)]
                for j in range(16):
                    r = g16 * 16 + j
                    ar = av16[j]
                    for g in range(8):
                        rows_v[r, pl.ds(g * 16, 16)] = (
                            rows_v[r, pl.ds(g * 16, 16)] * ar)
            pltpu.sync_copy(rows_v, acc_sh.at[idx_c_v], add=True)
            return carry

        lax.fori_loop(0, _NCHUNK2, body, 0)
        plsc.subcore_barrier()
        pltpu.sync_copy(acc_sh.at[pl.ds(sid * _TSLQ, _TSLQ)],
                        out_hbm.at[cid * 2 + p, pl.ds(sid * _TSLQ, _TSLQ)])
        plsc.subcore_barrier()


def _sc_msg(table_n, src2, dst2, ex2, denom):
    h = table_n.shape[1]
    mesh = plsc.VectorSubcoreMesh(core_axis_name="c", subcore_axis_name="s")
    f = pl.kernel(
        _sc_msg_body,
        mesh=mesh,
        out_type=jax.ShapeDtypeStruct((4, _NPQ, h), jnp.float32),
        scratch_types=[
            pltpu.VMEM((_NCHUNK2, _CH), jnp.int32),
            pltpu.VMEM((_NCHUNK2, _CH), jnp.int32),
            pltpu.VMEM((_CH,), jnp.int32),
            pltpu.VMEM((_NCHUNK2, _CH), jnp.float32),
            pltpu.VMEM((_CH,), jnp.float32),
            pltpu.VMEM((_CH,), jnp.float32),
            pltpu.VMEM((_CH, h), jnp.float32),
            pltpu.VMEM_SHARED((_NPQ, h), jnp.float32),
            pltpu.SemaphoreType.DMA,
            pltpu.SemaphoreType.DMA,
        ],
    )
    return f(table_n, src2, dst2, ex2, denom)


def _sc_gather2(table_i, table_j, dst, src):
    h = table_i.shape[1]
    dst3 = dst.reshape(_NW, _NCHUNK, _CH)
    src3 = src.reshape(_NW, _NCHUNK, _CH)
    mesh = plsc.VectorSubcoreMesh(core_axis_name="c", subcore_axis_name="s")
    f = pl.kernel(
        _sc_gather2_body,
        mesh=mesh,
        out_type=[
            jax.ShapeDtypeStruct((_E, h), jnp.float32),
            jax.ShapeDtypeStruct((_E, h), jnp.float32),
        ],
        scratch_types=[
            pltpu.VMEM((_NCHUNK, _CH), jnp.int32),
            pltpu.VMEM((_NCHUNK, _CH), jnp.int32),
            pltpu.VMEM((_CH, h), jnp.float32),
            pltpu.VMEM((_CH, h), jnp.float32),
            pltpu.SemaphoreType.DMA,
            pltpu.SemaphoreType.DMA,
        ],
    )
    return f(table_i, table_j, dst3, src3)


_PREC = jax.lax.Precision.DEFAULT


def _tc_proj_kernel(relu, x_ref, w_ref, b_ref, out_ref):
    x = x_ref[...]
    if relu:
        x = jnp.maximum(x, 0.0)
    out_ref[...] = (jnp.dot(x, w_ref[...], precision=_PREC,
                            preferred_element_type=jnp.float32) + b_ref[0])


def _tc_proj(x, w, b, relu):
    n, k = x.shape
    m = w.shape[1]
    blk = 2000
    return pl.pallas_call(
        functools.partial(_tc_proj_kernel, relu),
        grid=(n // blk,),
        in_specs=[
            pl.BlockSpec((blk, k), lambda i: (i, 0)),
            pl.BlockSpec((k, m), lambda i: (0, 0)),
            pl.BlockSpec(memory_space=pltpu.SMEM),
        ],
        out_specs=pl.BlockSpec((blk, m), lambda i: (i, 0)),
        out_shape=jax.ShapeDtypeStruct((n, m), jnp.float32),
    )(x, w, b)


def _tc_fused(g1, g2, c, we_c, av, we_next):
    e, h = g1.shape
    kc = c.shape[1]
    blk = 4000
    has_wec = we_c is not None
    has_next = we_next is not None
    av2 = av.reshape(h, 1)

    def kern(*refs):
        it = iter(refs)
        g1_ref = next(it)
        g2_ref = next(it)
        c_ref = next(it)
        we_ref = next(it) if has_wec else None
        av_ref = next(it)
        wn_ref = next(it) if has_next else None
        logit_ref = next(it)
        gmax_ref = next(it)
        cn_ref = next(it) if has_next else None
        i = pl.program_id(0)
        if has_wec:
            cterm = jnp.dot(c_ref[...], we_ref[...], precision=_PREC,
                            preferred_element_type=jnp.float32)
        else:
            cterm = c_ref[...]
        f = g1_ref[...] + g2_ref[...] + cterm
        e_act = jnp.where(f > 0, f, 0.2 * f)
        logits = jnp.dot(e_act, av_ref[...], precision=_PREC,
                         preferred_element_type=jnp.float32)
        logit_ref[...] = logits
        bmax = jnp.max(logits)

        @pl.when(i == 0)
        def _():
            gmax_ref[0, 0] = bmax

        @pl.when(i > 0)
        def _():
            gmax_ref[0, 0] = jnp.maximum(gmax_ref[0, 0], bmax)

        if has_next:
            cn_ref[...] = jnp.dot(f, wn_ref[...], precision=_PREC,
                                  preferred_element_type=jnp.float32)

    in_specs = [
        pl.BlockSpec((blk, h), lambda i: (i, 0)),
        pl.BlockSpec((blk, h), lambda i: (i, 0)),
        pl.BlockSpec((blk, kc), lambda i: (i, 0)),
    ]
    args = [g1, g2, c]
    if has_wec:
        in_specs.append(pl.BlockSpec((kc, h), lambda i: (0, 0)))
        args.append(we_c)
    in_specs.append(pl.BlockSpec((h, 1), lambda i: (0, 0)))
    args.append(av2)
    out_specs = [
        pl.BlockSpec((blk, 1), lambda i: (i, 0)),
        pl.BlockSpec((1, 1), lambda i: (0, 0), memory_space=pltpu.SMEM),
    ]
    out_shape = [
        jax.ShapeDtypeStruct((e, 1), jnp.float32),
        jax.ShapeDtypeStruct((1, 1), jnp.float32),
    ]
    if has_next:
        in_specs.append(pl.BlockSpec((h, h), lambda i: (0, 0)))
        args.append(we_next)
        out_specs.append(pl.BlockSpec((blk, h), lambda i: (i, 0)))
        out_shape.append(jax.ShapeDtypeStruct((e, h), jnp.float32))
    res = pl.pallas_call(
        kern, grid=(e // blk,), in_specs=in_specs,
        out_specs=out_specs, out_shape=out_shape,
    )(*args)
    c_next = res[2] if has_next else None
    return res[0].reshape(e), res[1][0, 0], c_next


def _layer(x, src, dst, c, we_c, Wn, Wi, Wj, av, n, We_next, relu_in):
    wcat = jnp.concatenate([Wi, Wj, Wn], axis=1)
    zb = jnp.zeros((1,), jnp.float32)
    tbl = _tc_proj(x, wcat, zb, relu_in)
    h = Wn.shape[1]
    xWi = tbl[:, :h]
    xWj = tbl[:, h:2 * h]
    xWn = tbl[:, 2 * h:]
    g1, g2 = _sc_gather2(xWi, xWj, dst, src)
    logits, gmax, c_next = _tc_fused(g1, g2, c, we_c, av, We_next)
    logits3 = logits.reshape(_NW, _NCHUNK, _CH)
    dst3 = dst.reshape(_NW, _NCHUNK, _CH)
    gmax16 = jnp.full((16,), gmax, jnp.float32)
    ex3, denom2 = _sc_denom(logits3, dst3, gmax16)
    denom = denom2[0] + denom2[1] + 1e-16
    src2 = src.reshape(_NS, _NCHUNK2, _CH)
    dst2 = dst.reshape(_NS, _NCHUNK2, _CH)
    ex2 = ex3.reshape(_NS, _NCHUNK2, _CH)
    out4 = _sc_msg(xWn, src2, dst2, ex2, denom)
    out = out4.reshape(4 * _NPQ, -1)[:n]
    return out, c_next


def kernel(x, edge_index, edge_attr, Wn1, Wi1, Wj1, We1, av1, Wn2, Wi2, Wj2, We2, av2, Wn3, Wi3, Wj3, We3, av3, Wc, bc):
    n = x.shape[0]
    src = edge_index[0]
    dst = edge_index[1]
    h, c2 = _layer(x, src, dst, edge_attr, We1, Wn1, Wi1, Wj1, av1, n, We2,
                   relu_in=False)
    h, c3 = _layer(h, src, dst, c2, None, Wn2, Wi2, Wj2, av2, n, We3,
                   relu_in=True)
    h, _ = _layer(h, src, dst, c3, None, Wn3, Wi3, Wj3, av3, n, None,
                  relu_in=True)
    return _tc_proj(h, Wc, bc, relu=True)


# speedup vs baseline: 1.2520x; 1.1478x over previous
import functools

import jax
import jax.numpy as jnp
from jax import lax
from jax.experimental import pallas as pl
from jax.experimental.pallas import tpu as pltpu
from jax.experimental.pallas import tpu_sc as plsc

_NC, _NS = 2, 16
_NW = _NC * _NS
_E = 320000
_BPW = _E // _NW
_CH = 80
_NCHUNK = _BPW // _CH


def _sc_gather2_body(ti_hbm, tj_hbm, dst_hbm, src_hbm, g1_hbm, g2_hbm,
                     idx_d_v, idx_s_v, rows1_v, rows2_v, sem1, sem2):
    wid = lax.axis_index("s") * _NC + lax.axis_index("c")
    base0 = wid * _BPW
    pltpu.sync_copy(dst_hbm.at[wid], idx_d_v)
    pltpu.sync_copy(src_hbm.at[wid], idx_s_v)

    def body(i, carry):
        base = base0 + i * _CH
        cp1 = pltpu.async_copy(ti_hbm.at[idx_d_v.at[i]], rows1_v, sem1)
        cp2 = pltpu.async_copy(tj_hbm.at[idx_s_v.at[i]], rows2_v, sem2)
        cp1.wait()
        cp2.wait()
        pltpu.sync_copy(rows1_v, g1_hbm.at[pl.ds(base, _CH)])
        pltpu.sync_copy(rows2_v, g2_hbm.at[pl.ds(base, _CH)])
        return carry

    lax.fori_loop(0, _NCHUNK, body, 0)


_NP = 10240
_TSL = _NP // _NS


def _sc_denom_body(logits_hbm, dst_hbm, gmax_hbm, ex_hbm, denom_hbm,
                   logit_v, ex_v, idx_v, gmax_v, zbuf_v, den_sh):
    sid = lax.axis_index("s")
    cid = lax.axis_index("c")
    wid = sid * _NC + cid
    z16 = jnp.zeros((16,), jnp.float32)
    for g in range(_TSL // 16):
        zbuf_v[pl.ds(g * 16, 16)] = z16
    pltpu.sync_copy(zbuf_v, den_sh.at[pl.ds(sid * _TSL, _TSL)])
    pltpu.sync_copy(logits_hbm.at[wid], logit_v)
    pltpu.sync_copy(dst_hbm.at[wid], idx_v)
    pltpu.sync_copy(gmax_hbm, gmax_v)
    gmax = gmax_v[...]
    plsc.subcore_barrier()

    def body(i, carry):
        for g in range(_CH // 16):
            lv = logit_v[i, pl.ds(g * 16, 16)]
            ex_v[i, pl.ds(g * 16, 16)] = jnp.exp(lv - gmax)
        pltpu.sync_copy(ex_v.at[i], den_sh.at[idx_v.at[i]], add=True)
        return carry

    lax.fori_loop(0, _NCHUNK, body, 0)
    pltpu.sync_copy(ex_v, ex_hbm.at[wid])
    plsc.subcore_barrier()
    pltpu.sync_copy(den_sh.at[pl.ds(sid * _TSL, _TSL)],
                    denom_hbm.at[cid, pl.ds(sid * _TSL, _TSL)])


def _sc_denom(logits3, dst3, gmax16):
    mesh = plsc.VectorSubcoreMesh(core_axis_name="c", subcore_axis_name="s")
    f = pl.kernel(
        _sc_denom_body,
        mesh=mesh,
        out_type=[
            jax.ShapeDtypeStruct((_NW, _NCHUNK, _CH), jnp.float32),
            jax.ShapeDtypeStruct((_NC, _NP), jnp.float32),
        ],
        scratch_types=[
            pltpu.VMEM((_NCHUNK, _CH), jnp.float32),
            pltpu.VMEM((_NCHUNK, _CH), jnp.float32),
            pltpu.VMEM((_NCHUNK, _CH), jnp.int32),
            pltpu.VMEM((16,), jnp.float32),
            pltpu.VMEM((_TSL,), jnp.float32),
            pltpu.VMEM_SHARED((_NP,), jnp.float32),
        ],
    )
    return f(logits3, dst3, gmax16)


_NP2 = _NP // 2
_TSL2 = _NP2 // _NS
_NCHUNK2 = _E // _NS // _CH


_NPQ = _NP // 4
_TSLQ = _NPQ // _NS


def _sc_msg_body(tn_hbm, src_hbm, dst_hbm, ex_hbm, denom_hbm, out_hbm,
                 idx_s_v, idx_d_v, idx_c_v, ex0_v, ex1_v, alpha_v,
                 d0_v, d1_v, rows0_v, rows1_v,
                 acc_sh, semr0, semr1, semd0, semd1):
    sid = lax.axis_index("s")
    cid = lax.axis_index("c")
    z16 = jnp.zeros((16,), jnp.float32)
    z16i = jnp.zeros((16,), jnp.int32)
    zf16 = jnp.zeros((16,), jnp.float32)
    rows = (rows0_v, rows1_v)
    dbuf = (d0_v, d1_v)
    ebuf = (ex0_v, ex1_v)
    semr = (semr0, semr1)
    semd = (semd0, semd1)
    pltpu.sync_copy(src_hbm.at[sid], idx_s_v)
    pltpu.sync_copy(dst_hbm.at[sid], idx_d_v)

    for p in range(2):
        for r in range(_CH):
            for g in range(8):
                rows0_v[r, pl.ds(g * 16, 16)] = z16
        for b in range(_TSLQ // _CH):
            pltpu.sync_copy(rows0_v, acc_sh.at[pl.ds(sid * _TSLQ + b * _CH, _CH)])
        plsc.subcore_barrier()
        lo = cid * _NP2 + p * _NPQ
        pltpu.async_copy(tn_hbm.at[idx_s_v.at[0]], rows0_v, semr0)
        pltpu.async_copy(denom_hbm.at[idx_d_v.at[0]], d0_v, semd0)
        pltpu.async_copy(ex_hbm.at[sid, 0], ex0_v, semd0)
        pltpu.async_copy(tn_hbm.at[idx_s_v.at[1]], rows1_v, semr1)
        pltpu.async_copy(denom_hbm.at[idx_d_v.at[1]], d1_v, semd1)
        pltpu.async_copy(ex_hbm.at[sid, 1], ex1_v, semd1)

        def pair(k, carry):
            for b in range(2):
                j = 2 * k + b
                pltpu.make_async_copy(
                    tn_hbm.at[idx_s_v.at[j]], rows[b], semr[b]).wait()
                pltpu.make_async_copy(
                    denom_hbm.at[idx_d_v.at[j]], dbuf[b], semd[b]).wait()
                pltpu.make_async_copy(
                    ex_hbm.at[sid, j], ebuf[b], semd[b]).wait()
                for g in range(_CH // 16):
                    d16 = dbuf[b][pl.ds(g * 16, 16)]
                    ex16 = ebuf[b][pl.ds(g * 16, 16)]
                    dv = idx_d_v[j, pl.ds(g * 16, 16)] - lo
                    inb = (dv >= 0) & (dv < _NPQ)
                    idx_c_v[pl.ds(g * 16, 16)] = jnp.where(inb, dv, z16i)
                    alpha_v[pl.ds(g * 16, 16)] = jnp.where(inb, ex16 / d16, zf16)
                for g16 in range(_CH // 16):
                    av16 = alpha_v[pl.ds(g16 * 16, 16)]
                    for jj in range(16):
                        r = g16 * 16 + jj
                        ar = av16[jj]
                        for g in range(8):
                            rows[b][r, pl.ds(g * 16, 16)] = (
                                rows[b][r, pl.ds(g * 16, 16)] * ar)
                pltpu.sync_copy(rows[b], acc_sh.at[idx_c_v], add=True)

                @pl.when(j + 2 < _NCHUNK2)
                def _():
                    pltpu.async_copy(
                        tn_hbm.at[idx_s_v.at[j + 2]], rows[b], semr[b])
                    pltpu.async_copy(
                        denom_hbm.at[idx_d_v.at[j + 2]], dbuf[b], semd[b])
                    pltpu.async_copy(ex_hbm.at[sid, j + 2], ebuf[b], semd[b])
            return carry

        lax.fori_loop(0, _NCHUNK2 // 2, pair, 0)
        plsc.subcore_barrier()
        pltpu.sync_copy(acc_sh.at[pl.ds(sid * _TSLQ, _TSLQ)],
                        out_hbm.at[cid * 2 + p, pl.ds(sid * _TSLQ, _TSLQ)])
        plsc.subcore_barrier()


def _sc_msg(table_n, src2, dst2, ex2, denom):
    h = table_n.shape[1]
    mesh = plsc.VectorSubcoreMesh(core_axis_name="c", subcore_axis_name="s")
    f = pl.kernel(
        _sc_msg_body,
        mesh=mesh,
        out_type=jax.ShapeDtypeStruct((4, _NPQ, h), jnp.float32),
        scratch_types=[
            pltpu.VMEM((_NCHUNK2, _CH), jnp.int32),
            pltpu.VMEM((_NCHUNK2, _CH), jnp.int32),
            pltpu.VMEM((_CH,), jnp.int32),
            pltpu.VMEM((_CH,), jnp.float32),
            pltpu.VMEM((_CH,), jnp.float32),
            pltpu.VMEM((_CH,), jnp.float32),
            pltpu.VMEM((_CH,), jnp.float32),
            pltpu.VMEM((_CH,), jnp.float32),
            pltpu.VMEM((_CH, h), jnp.float32),
            pltpu.VMEM((_CH, h), jnp.float32),
            pltpu.VMEM_SHARED((_NPQ, h), jnp.float32),
            pltpu.SemaphoreType.DMA,
            pltpu.SemaphoreType.DMA,
            pltpu.SemaphoreType.DMA,
            pltpu.SemaphoreType.DMA,
        ],
    )
    return f(table_n, src2, dst2, ex2, denom)


def _sc_gather2(table_i, table_j, dst, src):
    h = table_i.shape[1]
    dst3 = dst.reshape(_NW, _NCHUNK, _CH)
    src3 = src.reshape(_NW, _NCHUNK, _CH)
    mesh = plsc.VectorSubcoreMesh(core_axis_name="c", subcore_axis_name="s")
    f = pl.kernel(
        _sc_gather2_body,
        mesh=mesh,
        out_type=[
            jax.ShapeDtypeStruct((_E, h), jnp.float32),
            jax.ShapeDtypeStruct((_E, h), jnp.float32),
        ],
        scratch_types=[
            pltpu.VMEM((_NCHUNK, _CH), jnp.int32),
            pltpu.VMEM((_NCHUNK, _CH), jnp.int32),
            pltpu.VMEM((_CH, h), jnp.float32),
            pltpu.VMEM((_CH, h), jnp.float32),
            pltpu.SemaphoreType.DMA,
            pltpu.SemaphoreType.DMA,
        ],
    )
    return f(table_i, table_j, dst3, src3)


_PREC = jax.lax.Precision.DEFAULT


def _tc_proj_kernel(relu, x_ref, w_ref, b_ref, out_ref):
    x = x_ref[...]
    if relu:
        x = jnp.maximum(x, 0.0)
    out_ref[...] = (jnp.dot(x, w_ref[...], precision=_PREC,
                            preferred_element_type=jnp.float32) + b_ref[0])


def _tc_proj(x, w, b, relu):
    n, k = x.shape
    m = w.shape[1]
    blk = 2000
    return pl.pallas_call(
        functools.partial(_tc_proj_kernel, relu),
        grid=(n // blk,),
        in_specs=[
            pl.BlockSpec((blk, k), lambda i: (i, 0)),
            pl.BlockSpec((k, m), lambda i: (0, 0)),
            pl.BlockSpec(memory_space=pltpu.SMEM),
        ],
        out_specs=pl.BlockSpec((blk, m), lambda i: (i, 0)),
        out_shape=jax.ShapeDtypeStruct((n, m), jnp.float32),
    )(x, w, b)


def _tc_fused(g1, g2, c, we_c, av, we_next):
    e, h = g1.shape
    kc = c.shape[1]
    blk = 4000
    has_wec = we_c is not None
    has_next = we_next is not None
    av2 = av.reshape(h, 1)

    def kern(*refs):
        it = iter(refs)
        g1_ref = next(it)
        g2_ref = next(it)
        c_ref = next(it)
        we_ref = next(it) if has_wec else None
        av_ref = next(it)
        wn_ref = next(it) if has_next else None
        logit_ref = next(it)
        gmax_ref = next(it)
        cn_ref = next(it) if has_next else None
        i = pl.program_id(0)
        if has_wec:
            cterm = jnp.dot(c_ref[...], we_ref[...], precision=_PREC,
                            preferred_element_type=jnp.float32)
        else:
            cterm = c_ref[...]
        f = g1_ref[...] + g2_ref[...] + cterm
        e_act = jnp.where(f > 0, f, 0.2 * f)
        logits = jnp.dot(e_act, av_ref[...], precision=_PREC,
                         preferred_element_type=jnp.float32)
        logit_ref[...] = logits
        bmax = jnp.max(logits)

        @pl.when(i == 0)
        def _():
            gmax_ref[0, 0] = bmax

        @pl.when(i > 0)
        def _():
            gmax_ref[0, 0] = jnp.maximum(gmax_ref[0, 0], bmax)

        if has_next:
            cn_ref[...] = jnp.dot(f, wn_ref[...], precision=_PREC,
                                  preferred_element_type=jnp.float32)

    in_specs = [
        pl.BlockSpec((blk, h), lambda i: (i, 0)),
        pl.BlockSpec((blk, h), lambda i: (i, 0)),
        pl.BlockSpec((blk, kc), lambda i: (i, 0)),
    ]
    args = [g1, g2, c]
    if has_wec:
        in_specs.append(pl.BlockSpec((kc, h), lambda i: (0, 0)))
        args.append(we_c)
    in_specs.append(pl.BlockSpec((h, 1), lambda i: (0, 0)))
    args.append(av2)
    out_specs = [
        pl.BlockSpec((blk, 1), lambda i: (i, 0)),
        pl.BlockSpec((1, 1), lambda i: (0, 0), memory_space=pltpu.SMEM),
    ]
    out_shape = [
        jax.ShapeDtypeStruct((e, 1), jnp.float32),
        jax.ShapeDtypeStruct((1, 1), jnp.float32),
    ]
    if has_next:
        in_specs.append(pl.BlockSpec((h, h), lambda i: (0, 0)))
        args.append(we_next)
        out_specs.append(pl.BlockSpec((blk, h), lambda i: (i, 0)))
        out_shape.append(jax.ShapeDtypeStruct((e, h), jnp.float32))
    res = pl.pallas_call(
        kern, grid=(e // blk,), in_specs=in_specs,
        out_specs=out_specs, out_shape=out_shape,
    )(*args)
    c_next = res[2] if has_next else None
    return res[0].reshape(e), res[1][0, 0], c_next


def _layer(x, src, dst, c, we_c, Wn, Wi, Wj, av, n, We_next, relu_in):
    wcat = jnp.concatenate([Wi, Wj, Wn], axis=1)
    zb = jnp.zeros((1,), jnp.float32)
    tbl = _tc_proj(x, wcat, zb, relu_in)
    h = Wn.shape[1]
    xWi = tbl[:, :h]
    xWj = tbl[:, h:2 * h]
    xWn = tbl[:, 2 * h:]
    g1, g2 = _sc_gather2(xWi, xWj, dst, src)
    logits, gmax, c_next = _tc_fused(g1, g2, c, we_c, av, We_next)
    logits3 = logits.reshape(_NW, _NCHUNK, _CH)
    dst3 = dst.reshape(_NW, _NCHUNK, _CH)
    gmax16 = jnp.full((16,), gmax, jnp.float32)
    ex3, denom2 = _sc_denom(logits3, dst3, gmax16)
    denom = denom2[0] + denom2[1] + 1e-16
    src2 = src.reshape(_NS, _NCHUNK2, _CH)
    dst2 = dst.reshape(_NS, _NCHUNK2, _CH)
    ex2 = ex3.reshape(_NS, _NCHUNK2, _CH)
    out4 = _sc_msg(xWn, src2, dst2, ex2, denom)
    out = out4.reshape(4 * _NPQ, -1)[:n]
    return out, c_next


def kernel(x, edge_index, edge_attr, Wn1, Wi1, Wj1, We1, av1, Wn2, Wi2, Wj2, We2, av2, Wn3, Wi3, Wj3, We3, av3, Wc, bc):
    n = x.shape[0]
    src = edge_index[0]
    dst = edge_index[1]
    h, c2 = _layer(x, src, dst, edge_attr, We1, Wn1, Wi1, Wj1, av1, n, We2,
                   relu_in=False)
    h, c3 = _layer(h, src, dst, c2, None, Wn2, Wi2, Wj2, av2, n, We3,
                   relu_in=True)
    h, _ = _layer(h, src, dst, c3, None, Wn3, Wi3, Wj3, av3, n, None,
                  relu_in=True)
    return _tc_proj(h, Wc, bc, relu=True)


# speedup vs baseline: 1.3088x; 1.0453x over previous
import functools

import jax
import jax.numpy as jnp
from jax import lax
from jax.experimental import pallas as pl
from jax.experimental.pallas import tpu as pltpu
from jax.experimental.pallas import tpu_sc as plsc

_NC, _NS = 2, 16
_NW = _NC * _NS
_E = 320000
_BPW = _E // _NW
_CH = 80
_NCHUNK = _BPW // _CH


def _sc_gather2_body(ti_hbm, tj_hbm, dst_hbm, src_hbm, g1_hbm, g2_hbm,
                     idx_d_v, idx_s_v, r1a_v, r1b_v, r2a_v, r2b_v,
                     s1a, s1b, s2a, s2b):
    wid = lax.axis_index("s") * _NC + lax.axis_index("c")
    base0 = wid * _BPW
    pltpu.sync_copy(dst_hbm.at[wid], idx_d_v)
    pltpu.sync_copy(src_hbm.at[wid], idx_s_v)
    r1 = (r1a_v, r1b_v)
    r2 = (r2a_v, r2b_v)
    s1 = (s1a, s1b)
    s2 = (s2a, s2b)
    pltpu.async_copy(ti_hbm.at[idx_d_v.at[0]], r1a_v, s1a)
    pltpu.async_copy(tj_hbm.at[idx_s_v.at[0]], r2a_v, s2a)
    pltpu.async_copy(ti_hbm.at[idx_d_v.at[1]], r1b_v, s1b)
    pltpu.async_copy(tj_hbm.at[idx_s_v.at[1]], r2b_v, s2b)

    def pair(k, carry):
        for b in range(2):
            j = 2 * k + b
            base = base0 + j * _CH
            pltpu.make_async_copy(ti_hbm.at[idx_d_v.at[j]], r1[b], s1[b]).wait()
            pltpu.make_async_copy(tj_hbm.at[idx_s_v.at[j]], r2[b], s2[b]).wait()
            pltpu.sync_copy(r1[b], g1_hbm.at[pl.ds(base, _CH)])
            pltpu.sync_copy(r2[b], g2_hbm.at[pl.ds(base, _CH)])

            @pl.when(j + 2 < _NCHUNK)
            def _():
                pltpu.async_copy(ti_hbm.at[idx_d_v.at[j + 2]], r1[b], s1[b])
                pltpu.async_copy(tj_hbm.at[idx_s_v.at[j + 2]], r2[b], s2[b])
        return carry

    lax.fori_loop(0, _NCHUNK // 2, pair, 0)
    jt = _NCHUNK - 1
    pltpu.make_async_copy(ti_hbm.at[idx_d_v.at[jt]], r1a_v, s1a).wait()
    pltpu.make_async_copy(tj_hbm.at[idx_s_v.at[jt]], r2a_v, s2a).wait()
    pltpu.sync_copy(r1a_v, g1_hbm.at[pl.ds(base0 + jt * _CH, _CH)])
    pltpu.sync_copy(r2a_v, g2_hbm.at[pl.ds(base0 + jt * _CH, _CH)])


_NP = 10240
_TSL = _NP // _NS


def _sc_denom_body(logits_hbm, dst_hbm, gmax_hbm, ex_hbm, denom_hbm,
                   logit_v, ex_v, idx_v, gmax_v, zbuf_v, den_sh):
    sid = lax.axis_index("s")
    cid = lax.axis_index("c")
    wid = sid * _NC + cid
    z16 = jnp.zeros((16,), jnp.float32)
    for g in range(_TSL // 16):
        zbuf_v[pl.ds(g * 16, 16)] = z16
    pltpu.sync_copy(zbuf_v, den_sh.at[pl.ds(sid * _TSL, _TSL)])
    pltpu.sync_copy(logits_hbm.at[wid], logit_v)
    pltpu.sync_copy(dst_hbm.at[wid], idx_v)
    pltpu.sync_copy(gmax_hbm, gmax_v)
    gmax = gmax_v[...]
    plsc.subcore_barrier()

    def body(i, carry):
        for g in range(_CH // 16):
            lv = logit_v[i, pl.ds(g * 16, 16)]
            ex_v[i, pl.ds(g * 16, 16)] = jnp.exp(lv - gmax)
        pltpu.sync_copy(ex_v.at[i], den_sh.at[idx_v.at[i]], add=True)
        return carry

    lax.fori_loop(0, _NCHUNK, body, 0)
    pltpu.sync_copy(ex_v, ex_hbm.at[wid])
    plsc.subcore_barrier()
    pltpu.sync_copy(den_sh.at[pl.ds(sid * _TSL, _TSL)],
                    denom_hbm.at[cid, pl.ds(sid * _TSL, _TSL)])


def _sc_denom(logits3, dst3, gmax16):
    mesh = plsc.VectorSubcoreMesh(core_axis_name="c", subcore_axis_name="s")
    f = pl.kernel(
        _sc_denom_body,
        mesh=mesh,
        out_type=[
            jax.ShapeDtypeStruct((_NW, _NCHUNK, _CH), jnp.float32),
            jax.ShapeDtypeStruct((_NC, _NP), jnp.float32),
        ],
        scratch_types=[
            pltpu.VMEM((_NCHUNK, _CH), jnp.float32),
            pltpu.VMEM((_NCHUNK, _CH), jnp.float32),
            pltpu.VMEM((_NCHUNK, _CH), jnp.int32),
            pltpu.VMEM((16,), jnp.float32),
            pltpu.VMEM((_TSL,), jnp.float32),
            pltpu.VMEM_SHARED((_NP,), jnp.float32),
        ],
    )
    return f(logits3, dst3, gmax16)


_NP2 = _NP // 2
_TSL2 = _NP2 // _NS
_NCHUNK2 = _E // _NS // _CH


_NPQ = _NP // 4
_TSLQ = _NPQ // _NS


def _sc_msg_body(tn_hbm, src_hbm, dst_hbm, ex_hbm, denom_hbm, out_hbm,
                 idx_s_v, idx_d_v, idx_c_v, ex0_v, ex1_v, alpha_v,
                 d0_v, d1_v, rows0_v, rows1_v,
                 acc_sh, semr0, semr1, semd0, semd1):
    sid = lax.axis_index("s")
    cid = lax.axis_index("c")
    z16 = jnp.zeros((16,), jnp.float32)
    z16i = jnp.zeros((16,), jnp.int32)
    zf16 = jnp.zeros((16,), jnp.float32)
    rows = (rows0_v, rows1_v)
    dbuf = (d0_v, d1_v)
    ebuf = (ex0_v, ex1_v)
    semr = (semr0, semr1)
    semd = (semd0, semd1)
    pltpu.sync_copy(src_hbm.at[sid], idx_s_v)
    pltpu.sync_copy(dst_hbm.at[sid], idx_d_v)

    for p in range(2):
        for r in range(_CH):
            for g in range(8):
                rows0_v[r, pl.ds(g * 16, 16)] = z16
        for b in range(_TSLQ // _CH):
            pltpu.sync_copy(rows0_v, acc_sh.at[pl.ds(sid * _TSLQ + b * _CH, _CH)])
        plsc.subcore_barrier()
        lo = cid * _NP2 + p * _NPQ
        pltpu.async_copy(tn_hbm.at[idx_s_v.at[0]], rows0_v, semr0)
        pltpu.async_copy(denom_hbm.at[idx_d_v.at[0]], d0_v, semd0)
        pltpu.async_copy(ex_hbm.at[sid, 0], ex0_v, semd0)
        pltpu.async_copy(tn_hbm.at[idx_s_v.at[1]], rows1_v, semr1)
        pltpu.async_copy(denom_hbm.at[idx_d_v.at[1]], d1_v, semd1)
        pltpu.async_copy(ex_hbm.at[sid, 1], ex1_v, semd1)

        def pair(k, carry):
            for b in range(2):
                j = 2 * k + b
                pltpu.make_async_copy(
                    tn_hbm.at[idx_s_v.at[j]], rows[b], semr[b]).wait()
                pltpu.make_async_copy(
                    denom_hbm.at[idx_d_v.at[j]], dbuf[b], semd[b]).wait()
                pltpu.make_async_copy(
                    ex_hbm.at[sid, j], ebuf[b], semd[b]).wait()
                for g in range(_CH // 16):
                    d16 = dbuf[b][pl.ds(g * 16, 16)]
                    ex16 = ebuf[b][pl.ds(g * 16, 16)]
                    dv = idx_d_v[j, pl.ds(g * 16, 16)] - lo
                    inb = (dv >= 0) & (dv < _NPQ)
                    idx_c_v[pl.ds(g * 16, 16)] = jnp.where(inb, dv, z16i)
                    alpha_v[pl.ds(g * 16, 16)] = jnp.where(inb, ex16 / d16, zf16)
                for g16 in range(_CH // 16):
                    av16 = alpha_v[pl.ds(g16 * 16, 16)]
                    for jj in range(16):
                        r = g16 * 16 + jj
                        ar = av16[jj]
                        for g in range(8):
                            rows[b][r, pl.ds(g * 16, 16)] = (
                                rows[b][r, pl.ds(g * 16, 16)] * ar)
                pltpu.sync_copy(rows[b], acc_sh.at[idx_c_v], add=True)

                @pl.when(j + 2 < _NCHUNK2)
                def _():
                    pltpu.async_copy(
                        tn_hbm.at[idx_s_v.at[j + 2]], rows[b], semr[b])
                    pltpu.async_copy(
                        denom_hbm.at[idx_d_v.at[j + 2]], dbuf[b], semd[b])
                    pltpu.async_copy(ex_hbm.at[sid, j + 2], ebuf[b], semd[b])
            return carry

        lax.fori_loop(0, _NCHUNK2 // 2, pair, 0)
        plsc.subcore_barrier()
        pltpu.sync_copy(acc_sh.at[pl.ds(sid * _TSLQ, _TSLQ)],
                        out_hbm.at[cid * 2 + p, pl.ds(sid * _TSLQ, _TSLQ)])
        plsc.subcore_barrier()


def _sc_msg(table_n, src2, dst2, ex2, denom):
    h = table_n.shape[1]
    mesh = plsc.VectorSubcoreMesh(core_axis_name="c", subcore_axis_name="s")
    f = pl.kernel(
        _sc_msg_body,
        mesh=mesh,
        out_type=jax.ShapeDtypeStruct((4, _NPQ, h), jnp.float32),
        scratch_types=[
            pltpu.VMEM((_NCHUNK2, _CH), jnp.int32),
            pltpu.VMEM((_NCHUNK2, _CH), jnp.int32),
            pltpu.VMEM((_CH,), jnp.int32),
            pltpu.VMEM((_CH,), jnp.float32),
            pltpu.VMEM((_CH,), jnp.float32),
            pltpu.VMEM((_CH,), jnp.float32),
            pltpu.VMEM((_CH,), jnp.float32),
            pltpu.VMEM((_CH,), jnp.float32),
            pltpu.VMEM((_CH, h), jnp.float32),
            pltpu.VMEM((_CH, h), jnp.float32),
            pltpu.VMEM_SHARED((_NPQ, h), jnp.float32),
            pltpu.SemaphoreType.DMA,
            pltpu.SemaphoreType.DMA,
            pltpu.SemaphoreType.DMA,
            pltpu.SemaphoreType.DMA,
        ],
    )
    return f(table_n, src2, dst2, ex2, denom)


def _sc_gather2(table_i, table_j, dst, src):
    h = table_i.shape[1]
    dst3 = dst.reshape(_NW, _NCHUNK, _CH)
    src3 = src.reshape(_NW, _NCHUNK, _CH)
    mesh = plsc.VectorSubcoreMesh(core_axis_name="c", subcore_axis_name="s")
    f = pl.kernel(
        _sc_gather2_body,
        mesh=mesh,
        out_type=[
            jax.ShapeDtypeStruct((_E, h), jnp.float32),
            jax.ShapeDtypeStruct((_E, h), jnp.float32),
        ],
        scratch_types=[
            pltpu.VMEM((_NCHUNK, _CH), jnp.int32),
            pltpu.VMEM((_NCHUNK, _CH), jnp.int32),
            pltpu.VMEM((_CH, h), jnp.float32),
            pltpu.VMEM((_CH, h), jnp.float32),
            pltpu.VMEM((_CH, h), jnp.float32),
            pltpu.VMEM((_CH, h), jnp.float32),
            pltpu.SemaphoreType.DMA,
            pltpu.SemaphoreType.DMA,
            pltpu.SemaphoreType.DMA,
            pltpu.SemaphoreType.DMA,
        ],
    )
    return f(table_i, table_j, dst3, src3)


_PREC = jax.lax.Precision.DEFAULT


def _tc_proj_kernel(relu, x_ref, w_ref, b_ref, out_ref):
    x = x_ref[...]
    if relu:
        x = jnp.maximum(x, 0.0)
    out_ref[...] = (jnp.dot(x, w_ref[...], precision=_PREC,
                            preferred_element_type=jnp.float32) + b_ref[0])


def _tc_proj(x, w, b, relu):
    n, k = x.shape
    m = w.shape[1]
    blk = 2000
    return pl.pallas_call(
        functools.partial(_tc_proj_kernel, relu),
        grid=(n // blk,),
        in_specs=[
            pl.BlockSpec((blk, k), lambda i: (i, 0)),
            pl.BlockSpec((k, m), lambda i: (0, 0)),
            pl.BlockSpec(memory_space=pltpu.SMEM),
        ],
        out_specs=pl.BlockSpec((blk, m), lambda i: (i, 0)),
        out_shape=jax.ShapeDtypeStruct((n, m), jnp.float32),
    )(x, w, b)


def _tc_fused(g1, g2, c, we_c, av, we_next):
    e, h = g1.shape
    kc = c.shape[1]
    blk = 4000
    has_wec = we_c is not None
    has_next = we_next is not None
    av2 = av.reshape(h, 1)

    def kern(*refs):
        it = iter(refs)
        g1_ref = next(it)
        g2_ref = next(it)
        c_ref = next(it)
        we_ref = next(it) if has_wec else None
        av_ref = next(it)
        wn_ref = next(it) if has_next else None
        logit_ref = next(it)
        gmax_ref = next(it)
        cn_ref = next(it) if has_next else None
        i = pl.program_id(0)
        if has_wec:
            cterm = jnp.dot(c_ref[...], we_ref[...], precision=_PREC,
                            preferred_element_type=jnp.float32)
        else:
            cterm = c_ref[...]
        f = g1_ref[...] + g2_ref[...] + cterm
        e_act = jnp.where(f > 0, f, 0.2 * f)
        logits = jnp.dot(e_act, av_ref[...], precision=_PREC,
                         preferred_element_type=jnp.float32)
        logit_ref[...] = logits
        bmax = jnp.max(logits)

        @pl.when(i == 0)
        def _():
            gmax_ref[0, 0] = bmax

        @pl.when(i > 0)
        def _():
            gmax_ref[0, 0] = jnp.maximum(gmax_ref[0, 0], bmax)

        if has_next:
            cn_ref[...] = jnp.dot(f, wn_ref[...], precision=_PREC,
                                  preferred_element_type=jnp.float32)

    in_specs = [
        pl.BlockSpec((blk, h), lambda i: (i, 0)),
        pl.BlockSpec((blk, h), lambda i: (i, 0)),
        pl.BlockSpec((blk, kc), lambda i: (i, 0)),
    ]
    args = [g1, g2, c]
    if has_wec:
        in_specs.append(pl.BlockSpec((kc, h), lambda i: (0, 0)))
        args.append(we_c)
    in_specs.append(pl.BlockSpec((h, 1), lambda i: (0, 0)))
    args.append(av2)
    out_specs = [
        pl.BlockSpec((blk, 1), lambda i: (i, 0)),
        pl.BlockSpec((1, 1), lambda i: (0, 0), memory_space=pltpu.SMEM),
    ]
    out_shape = [
        jax.ShapeDtypeStruct((e, 1), jnp.float32),
        jax.ShapeDtypeStruct((1, 1), jnp.float32),
    ]
    if has_next:
        in_specs.append(pl.BlockSpec((h, h), lambda i: (0, 0)))
        args.append(we_next)
        out_specs.append(pl.BlockSpec((blk, h), lambda i: (i, 0)))
        out_shape.append(jax.ShapeDtypeStruct((e, h), jnp.float32))
    res = pl.pallas_call(
        kern, grid=(e // blk,), in_specs=in_specs,
        out_specs=out_specs, out_shape=out_shape,
    )(*args)
    c_next = res[2] if has_next else None
    return res[0].reshape(e), res[1][0, 0], c_next


def _layer(x, src, dst, c, we_c, Wn, Wi, Wj, av, n, We_next, relu_in):
    wcat = jnp.concatenate([Wi, Wj, Wn], axis=1)
    zb = jnp.zeros((1,), jnp.float32)
    tbl = _tc_proj(x, wcat, zb, relu_in)
    h = Wn.shape[1]
    xWi = tbl[:, :h]
    xWj = tbl[:, h:2 * h]
    xWn = tbl[:, 2 * h:]
    g1, g2 = _sc_gather2(xWi, xWj, dst, src)
    logits, gmax, c_next = _tc_fused(g1, g2, c, we_c, av, We_next)
    logits3 = logits.reshape(_NW, _NCHUNK, _CH)
    dst3 = dst.reshape(_NW, _NCHUNK, _CH)
    gmax16 = jnp.full((16,), gmax, jnp.float32)
    ex3, denom2 = _sc_denom(logits3, dst3, gmax16)
    denom = denom2[0] + denom2[1] + 1e-16
    src2 = src.reshape(_NS, _NCHUNK2, _CH)
    dst2 = dst.reshape(_NS, _NCHUNK2, _CH)
    ex2 = ex3.reshape(_NS, _NCHUNK2, _CH)
    out4 = _sc_msg(xWn, src2, dst2, ex2, denom)
    out = out4.reshape(4 * _NPQ, -1)[:n]
    return out, c_next


def kernel(x, edge_index, edge_attr, Wn1, Wi1, Wj1, We1, av1, Wn2, Wi2, Wj2, We2, av2, Wn3, Wi3, Wj3, We3, av3, Wc, bc):
    n = x.shape[0]
    src = edge_index[0]
    dst = edge_index[1]
    h, c2 = _layer(x, src, dst, edge_attr, We1, Wn1, Wi1, Wj1, av1, n, We2,
                   relu_in=False)
    h, c3 = _layer(h, src, dst, c2, None, Wn2, Wi2, Wj2, av2, n, We3,
                   relu_in=True)
    h, _ = _layer(h, src, dst, c3, None, Wn3, Wi3, Wj3, av3, n, None,
                  relu_in=True)
    return _tc_proj(h, Wc, bc, relu=True)


# speedup vs baseline: 1.3116x; 1.0021x over previous
import functools

import jax
import jax.numpy as jnp
from jax import lax
from jax.experimental import pallas as pl
from jax.experimental.pallas import tpu as pltpu
from jax.experimental.pallas import tpu_sc as plsc

_NC, _NS = 2, 16
_NW = _NC * _NS
_E = 320000
_BPW = _E // _NW
_CH = 80
_NCHUNK = _BPW // _CH


def _sc_gather2_body(ti_hbm, tj_hbm, dst_hbm, src_hbm, g1_hbm, g2_hbm,
                     idx_d_v, idx_s_v, r1a_v, r1b_v, r2a_v, r2b_v,
                     s1a, s1b, s2a, s2b):
    wid = lax.axis_index("s") * _NC + lax.axis_index("c")
    base0 = wid * _BPW
    pltpu.sync_copy(dst_hbm.at[wid], idx_d_v)
    pltpu.sync_copy(src_hbm.at[wid], idx_s_v)
    r1 = (r1a_v, r1b_v)
    r2 = (r2a_v, r2b_v)
    s1 = (s1a, s1b)
    s2 = (s2a, s2b)
    pltpu.async_copy(ti_hbm.at[idx_d_v.at[0]], r1a_v, s1a)
    pltpu.async_copy(tj_hbm.at[idx_s_v.at[0]], r2a_v, s2a)
    pltpu.async_copy(ti_hbm.at[idx_d_v.at[1]], r1b_v, s1b)
    pltpu.async_copy(tj_hbm.at[idx_s_v.at[1]], r2b_v, s2b)

    def pair(k, carry):
        for b in range(2):
            j = 2 * k + b
            base = base0 + j * _CH
            pltpu.make_async_copy(ti_hbm.at[idx_d_v.at[j]], r1[b], s1[b]).wait()
            pltpu.make_async_copy(tj_hbm.at[idx_s_v.at[j]], r2[b], s2[b]).wait()
            pltpu.sync_copy(r1[b], g1_hbm.at[pl.ds(base, _CH)])
            pltpu.sync_copy(r2[b], g2_hbm.at[pl.ds(base, _CH)])

            @pl.when(j + 2 < _NCHUNK)
            def _():
                pltpu.async_copy(ti_hbm.at[idx_d_v.at[j + 2]], r1[b], s1[b])
                pltpu.async_copy(tj_hbm.at[idx_s_v.at[j + 2]], r2[b], s2[b])
        return carry

    lax.fori_loop(0, _NCHUNK // 2, pair, 0)
    jt = _NCHUNK - 1
    pltpu.make_async_copy(ti_hbm.at[idx_d_v.at[jt]], r1a_v, s1a).wait()
    pltpu.make_async_copy(tj_hbm.at[idx_s_v.at[jt]], r2a_v, s2a).wait()
    pltpu.sync_copy(r1a_v, g1_hbm.at[pl.ds(base0 + jt * _CH, _CH)])
    pltpu.sync_copy(r2a_v, g2_hbm.at[pl.ds(base0 + jt * _CH, _CH)])


_NP = 10240
_TSL = _NP // _NS


def _sc_denom_body(logits_hbm, dst_hbm, gmax_hbm, ex_hbm, denom_hbm,
                   logit_v, ex_v, idx_v, gmax_v, zbuf_v, den_sh):
    sid = lax.axis_index("s")
    cid = lax.axis_index("c")
    wid = sid * _NC + cid
    z16 = jnp.zeros((16,), jnp.float32)
    for g in range(_TSL // 16):
        zbuf_v[pl.ds(g * 16, 16)] = z16
    pltpu.sync_copy(zbuf_v, den_sh.at[pl.ds(sid * _TSL, _TSL)])
    pltpu.sync_copy(logits_hbm.at[wid], logit_v)
    pltpu.sync_copy(dst_hbm.at[wid], idx_v)
    pltpu.sync_copy(gmax_hbm, gmax_v)
    gmax = gmax_v[...]
    plsc.subcore_barrier()

    def body(i, carry):
        for g in range(_CH // 16):
            lv = logit_v[i, pl.ds(g * 16, 16)]
            ex_v[i, pl.ds(g * 16, 16)] = jnp.exp(lv - gmax)
        pltpu.sync_copy(ex_v.at[i], den_sh.at[idx_v.at[i]], add=True)
        return carry

    lax.fori_loop(0, _NCHUNK, body, 0)
    pltpu.sync_copy(ex_v, ex_hbm.at[wid])
    plsc.subcore_barrier()
    pltpu.sync_copy(den_sh.at[pl.ds(sid * _TSL, _TSL)],
                    denom_hbm.at[cid, pl.ds(sid * _TSL, _TSL)])


def _sc_denom(logits3, dst3, gmax16):
    mesh = plsc.VectorSubcoreMesh(core_axis_name="c", subcore_axis_name="s")
    f = pl.kernel(
        _sc_denom_body,
        mesh=mesh,
        out_type=[
            jax.ShapeDtypeStruct((_NW, _NCHUNK, _CH), jnp.float32),
            jax.ShapeDtypeStruct((_NC, _NP), jnp.float32),
        ],
        scratch_types=[
            pltpu.VMEM((_NCHUNK, _CH), jnp.float32),
            pltpu.VMEM((_NCHUNK, _CH), jnp.float32),
            pltpu.VMEM((_NCHUNK, _CH), jnp.int32),
            pltpu.VMEM((16,), jnp.float32),
            pltpu.VMEM((_TSL,), jnp.float32),
            pltpu.VMEM_SHARED((_NP,), jnp.float32),
        ],
    )
    return f(logits3, dst3, gmax16)


_NP2 = _NP // 2
_TSL2 = _NP2 // _NS
_NCHUNK2 = _E // _NS // _CH


_NPQ = _NP // 4
_TSLQ = _NPQ // _NS


def _sc_msg_body(tn_hbm, src_hbm, dst_hbm, ex_hbm, denom_hbm, out_hbm,
                 idx_s_v, idx_d_v, idx_c_v, ex0_v, ex1_v, alpha_v,
                 d0_v, d1_v, rows0_v, rows1_v,
                 acc_sh, semr0, semr1, semd0, semd1):
    sid = lax.axis_index("s")
    cid = lax.axis_index("c")
    z16 = jnp.zeros((16,), jnp.float32)
    z16i = jnp.zeros((16,), jnp.int32)
    zf16 = jnp.zeros((16,), jnp.float32)
    rows = (rows0_v, rows1_v)
    dbuf = (d0_v, d1_v)
    ebuf = (ex0_v, ex1_v)
    semr = (semr0, semr1)
    semd = (semd0, semd1)
    pltpu.sync_copy(src_hbm.at[sid], idx_s_v)
    pltpu.sync_copy(dst_hbm.at[sid], idx_d_v)

    for p in range(2):
        for r in range(_CH):
            for g in range(8):
                rows0_v[r, pl.ds(g * 16, 16)] = z16
        for b in range(_TSLQ // _CH):
            pltpu.sync_copy(rows0_v, acc_sh.at[pl.ds(sid * _TSLQ + b * _CH, _CH)])
        plsc.subcore_barrier()
        lo = cid * _NP2 + p * _NPQ
        pltpu.async_copy(tn_hbm.at[idx_s_v.at[0]], rows0_v, semr0)
        pltpu.async_copy(denom_hbm.at[idx_d_v.at[0]], d0_v, semd0)
        pltpu.async_copy(ex_hbm.at[sid, 0], ex0_v, semd0)
        pltpu.async_copy(tn_hbm.at[idx_s_v.at[1]], rows1_v, semr1)
        pltpu.async_copy(denom_hbm.at[idx_d_v.at[1]], d1_v, semd1)
        pltpu.async_copy(ex_hbm.at[sid, 1], ex1_v, semd1)

        def pair(k, carry):
            for b in range(2):
                j = 2 * k + b
                pltpu.make_async_copy(
                    tn_hbm.at[idx_s_v.at[j]], rows[b], semr[b]).wait()
                pltpu.make_async_copy(
                    denom_hbm.at[idx_d_v.at[j]], dbuf[b], semd[b]).wait()
                pltpu.make_async_copy(
                    ex_hbm.at[sid, j], ebuf[b], semd[b]).wait()
                for g in range(_CH // 16):
                    d16 = dbuf[b][pl.ds(g * 16, 16)]
                    ex16 = ebuf[b][pl.ds(g * 16, 16)]
                    dv = idx_d_v[j, pl.ds(g * 16, 16)] - lo
                    inb = (dv >= 0) & (dv < _NPQ)
                    idx_c_v[pl.ds(g * 16, 16)] = jnp.where(inb, dv, z16i)
                    alpha_v[pl.ds(g * 16, 16)] = jnp.where(inb, ex16 / d16, zf16)
                for g16 in range(_CH // 16):
                    av16 = alpha_v[pl.ds(g16 * 16, 16)]
                    for jj in range(16):
                        r = g16 * 16 + jj
                        ar = av16[jj]
                        for g in range(8):
                            rows[b][r, pl.ds(g * 16, 16)] = (
                                rows[b][r, pl.ds(g * 16, 16)] * ar)
                pltpu.sync_copy(rows[b], acc_sh.at[idx_c_v], add=True)

                @pl.when(j + 2 < _NCHUNK2)
                def _():
                    pltpu.async_copy(
                        tn_hbm.at[idx_s_v.at[j + 2]], rows[b], semr[b])
                    pltpu.async_copy(
                        denom_hbm.at[idx_d_v.at[j + 2]], dbuf[b], semd[b])
                    pltpu.async_copy(ex_hbm.at[sid, j + 2], ebuf[b], semd[b])
            return carry

        lax.fori_loop(0, _NCHUNK2 // 2, pair, 0)
        plsc.subcore_barrier()
        pltpu.sync_copy(acc_sh.at[pl.ds(sid * _TSLQ, _TSLQ)],
                        out_hbm.at[cid * 2 + p, pl.ds(sid * _TSLQ, _TSLQ)])
        plsc.subcore_barrier()


def _sc_msg(table_n, src2, dst2, ex2, denom):
    h = table_n.shape[1]
    mesh = plsc.VectorSubcoreMesh(core_axis_name="c", subcore_axis_name="s")
    f = pl.kernel(
        _sc_msg_body,
        mesh=mesh,
        out_type=jax.ShapeDtypeStruct((4, _NPQ, h), jnp.float32),
        scratch_types=[
            pltpu.VMEM((_NCHUNK2, _CH), jnp.int32),
            pltpu.VMEM((_NCHUNK2, _CH), jnp.int32),
            pltpu.VMEM((_CH,), jnp.int32),
            pltpu.VMEM((_CH,), jnp.float32),
            pltpu.VMEM((_CH,), jnp.float32),
            pltpu.VMEM((_CH,), jnp.float32),
            pltpu.VMEM((_CH,), jnp.float32),
            pltpu.VMEM((_CH,), jnp.float32),
            pltpu.VMEM((_CH, h), jnp.float32),
            pltpu.VMEM((_CH, h), jnp.float32),
            pltpu.VMEM_SHARED((_NPQ, h), jnp.float32),
            pltpu.SemaphoreType.DMA,
            pltpu.SemaphoreType.DMA,
            pltpu.SemaphoreType.DMA,
            pltpu.SemaphoreType.DMA,
        ],
    )
    return f(table_n, src2, dst2, ex2, denom)


def _sc_gather2(table_i, table_j, dst, src):
    h = table_i.shape[1]
    dst3 = dst.reshape(_NW, _NCHUNK, _CH)
    src3 = src.reshape(_NW, _NCHUNK, _CH)
    mesh = plsc.VectorSubcoreMesh(core_axis_name="c", subcore_axis_name="s")
    f = pl.kernel(
        _sc_gather2_body,
        mesh=mesh,
        out_type=[
            jax.ShapeDtypeStruct((_E, h), jnp.float32),
            jax.ShapeDtypeStruct((_E, h), jnp.float32),
        ],
        scratch_types=[
            pltpu.VMEM((_NCHUNK, _CH), jnp.int32),
            pltpu.VMEM((_NCHUNK, _CH), jnp.int32),
            pltpu.VMEM((_CH, h), jnp.float32),
            pltpu.VMEM((_CH, h), jnp.float32),
            pltpu.VMEM((_CH, h), jnp.float32),
            pltpu.VMEM((_CH, h), jnp.float32),
            pltpu.SemaphoreType.DMA,
            pltpu.SemaphoreType.DMA,
            pltpu.SemaphoreType.DMA,
            pltpu.SemaphoreType.DMA,
        ],
    )
    return f(table_i, table_j, dst3, src3)


_PREC = jax.lax.Precision.DEFAULT


def _tc_proj_kernel(relu, x_ref, w_ref, b_ref, out_ref):
    x = x_ref[...]
    if relu:
        x = jnp.maximum(x, 0.0)
    out_ref[...] = (jnp.dot(x, w_ref[...], precision=_PREC,
                            preferred_element_type=jnp.float32) + b_ref[0])


def _tc_proj(x, w, b, relu):
    n, k = x.shape
    m = w.shape[1]
    blk = 2000
    return pl.pallas_call(
        functools.partial(_tc_proj_kernel, relu),
        grid=(n // blk,),
        in_specs=[
            pl.BlockSpec((blk, k), lambda i: (i, 0)),
            pl.BlockSpec((k, m), lambda i: (0, 0)),
            pl.BlockSpec(memory_space=pltpu.SMEM),
        ],
        out_specs=pl.BlockSpec((blk, m), lambda i: (i, 0)),
        out_shape=jax.ShapeDtypeStruct((n, m), jnp.float32),
    )(x, w, b)


def _tc_fused(g1, g2, c, we_c, av, we_next):
    e, h = g1.shape
    kc = c.shape[1]
    blk = 8000
    has_wec = we_c is not None
    has_next = we_next is not None
    av2 = av.reshape(h, 1)

    def kern(*refs):
        it = iter(refs)
        g1_ref = next(it)
        g2_ref = next(it)
        c_ref = next(it)
        we_ref = next(it) if has_wec else None
        av_ref = next(it)
        wn_ref = next(it) if has_next else None
        logit_ref = next(it)
        gmax_ref = next(it)
        cn_ref = next(it) if has_next else None
        i = pl.program_id(0)
        if has_wec:
            cterm = jnp.dot(c_ref[...], we_ref[...], precision=_PREC,
                            preferred_element_type=jnp.float32)
        else:
            cterm = c_ref[...]
        f = g1_ref[...] + g2_ref[...] + cterm
        e_act = jnp.where(f > 0, f, 0.2 * f)
        logits = jnp.dot(e_act, av_ref[...], precision=_PREC,
                         preferred_element_type=jnp.float32)
        logit_ref[...] = logits
        bmax = jnp.max(logits)

        @pl.when(i == 0)
        def _():
            gmax_ref[0, 0] = bmax

        @pl.when(i > 0)
        def _():
            gmax_ref[0, 0] = jnp.maximum(gmax_ref[0, 0], bmax)

        if has_next:
            cn_ref[...] = jnp.dot(f, wn_ref[...], precision=_PREC,
                                  preferred_element_type=jnp.float32)

    in_specs = [
        pl.BlockSpec((blk, h), lambda i: (i, 0)),
        pl.BlockSpec((blk, h), lambda i: (i, 0)),
        pl.BlockSpec((blk, kc), lambda i: (i, 0)),
    ]
    args = [g1, g2, c]
    if has_wec:
        in_specs.append(pl.BlockSpec((kc, h), lambda i: (0, 0)))
        args.append(we_c)
    in_specs.append(pl.BlockSpec((h, 1), lambda i: (0, 0)))
    args.append(av2)
    out_specs = [
        pl.BlockSpec((blk, 1), lambda i: (i, 0)),
        pl.BlockSpec((1, 1), lambda i: (0, 0), memory_space=pltpu.SMEM),
    ]
    out_shape = [
        jax.ShapeDtypeStruct((e, 1), jnp.float32),
        jax.ShapeDtypeStruct((1, 1), jnp.float32),
    ]
    if has_next:
        in_specs.append(pl.BlockSpec((h, h), lambda i: (0, 0)))
        args.append(we_next)
        out_specs.append(pl.BlockSpec((blk, h), lambda i: (i, 0)))
        out_shape.append(jax.ShapeDtypeStruct((e, h), jnp.float32))
    res = pl.pallas_call(
        kern, grid=(e // blk,), in_specs=in_specs,
        out_specs=out_specs, out_shape=out_shape,
    )(*args)
    c_next = res[2] if has_next else None
    return res[0].reshape(e), res[1][0, 0], c_next


def _layer(x, src, dst, c, we_c, Wn, Wi, Wj, av, n, We_next, relu_in):
    wcat = jnp.concatenate([Wi, Wj, Wn], axis=1)
    zb = jnp.zeros((1,), jnp.float32)
    tbl = _tc_proj(x, wcat, zb, relu_in)
    h = Wn.shape[1]
    xWi = tbl[:, :h]
    xWj = tbl[:, h:2 * h]
    xWn = tbl[:, 2 * h:]
    g1, g2 = _sc_gather2(xWi, xWj, dst, src)
    logits, gmax, c_next = _tc_fused(g1, g2, c, we_c, av, We_next)
    logits3 = logits.reshape(_NW, _NCHUNK, _CH)
    dst3 = dst.reshape(_NW, _NCHUNK, _CH)
    gmax16 = jnp.full((16,), gmax, jnp.float32)
    ex3, denom2 = _sc_denom(logits3, dst3, gmax16)
    denom = denom2[0] + denom2[1] + 1e-16
    src2 = src.reshape(_NS, _NCHUNK2, _CH)
    dst2 = dst.reshape(_NS, _NCHUNK2, _CH)
    ex2 = ex3.reshape(_NS, _NCHUNK2, _CH)
    out4 = _sc_msg(xWn, src2, dst2, ex2, denom)
    out = out4.reshape(4 * _NPQ, -1)[:n]
    return out, c_next


def kernel(x, edge_index, edge_attr, Wn1, Wi1, Wj1, We1, av1, Wn2, Wi2, Wj2, We2, av2, Wn3, Wi3, Wj3, We3, av3, Wc, bc):
    n = x.shape[0]
    src = edge_index[0]
    dst = edge_index[1]
    h, c2 = _layer(x, src, dst, edge_attr, We1, Wn1, Wi1, Wj1, av1, n, We2,
                   relu_in=False)
    h, c3 = _layer(h, src, dst, c2, None, Wn2, Wi2, Wj2, av2, n, We3,
                   relu_in=True)
    h, _ = _layer(h, src, dst, c3, None, Wn3, Wi3, Wj3, av3, n, None,
                  relu_in=True)
    return _tc_proj(h, Wc, bc, relu=True)


# speedup vs baseline: 1.9980x; 1.5233x over previous
import functools

import jax
import jax.numpy as jnp
from jax import lax
from jax.experimental import pallas as pl
from jax.experimental.pallas import tpu as pltpu
from jax.experimental.pallas import tpu_sc as plsc

_NC, _NS = 2, 16
_NW = _NC * _NS
_E = 320000
_BPW = _E // _NW
_CH = 80
_NCHUNK = _BPW // _CH


def _sc_gather2_body(ti_hbm, tj_hbm, dst_hbm, src_hbm, g1_hbm, g2_hbm,
                     idx_d_v, idx_s_v, r1a_v, r1b_v, r2a_v, r2b_v,
                     s1a, s1b, s2a, s2b):
    wid = lax.axis_index("s") * _NC + lax.axis_index("c")
    base0 = wid * _BPW
    pltpu.sync_copy(dst_hbm.at[wid], idx_d_v)
    pltpu.sync_copy(src_hbm.at[wid], idx_s_v)
    r1 = (r1a_v, r1b_v)
    r2 = (r2a_v, r2b_v)
    s1 = (s1a, s1b)
    s2 = (s2a, s2b)
    pltpu.async_copy(ti_hbm.at[idx_d_v.at[0]], r1a_v, s1a)
    pltpu.async_copy(tj_hbm.at[idx_s_v.at[0]], r2a_v, s2a)
    pltpu.async_copy(ti_hbm.at[idx_d_v.at[1]], r1b_v, s1b)
    pltpu.async_copy(tj_hbm.at[idx_s_v.at[1]], r2b_v, s2b)

    def pair(k, carry):
        for b in range(2):
            j = 2 * k + b
            base = base0 + j * _CH
            pltpu.make_async_copy(ti_hbm.at[idx_d_v.at[j]], r1[b], s1[b]).wait()
            pltpu.make_async_copy(tj_hbm.at[idx_s_v.at[j]], r2[b], s2[b]).wait()
            pltpu.sync_copy(r1[b], g1_hbm.at[pl.ds(base, _CH)])
            pltpu.sync_copy(r2[b], g2_hbm.at[pl.ds(base, _CH)])

            @pl.when(j + 2 < _NCHUNK)
            def _():
                pltpu.async_copy(ti_hbm.at[idx_d_v.at[j + 2]], r1[b], s1[b])
                pltpu.async_copy(tj_hbm.at[idx_s_v.at[j + 2]], r2[b], s2[b])
        return carry

    lax.fori_loop(0, _NCHUNK // 2, pair, 0)
    jt = _NCHUNK - 1
    pltpu.make_async_copy(ti_hbm.at[idx_d_v.at[jt]], r1a_v, s1a).wait()
    pltpu.make_async_copy(tj_hbm.at[idx_s_v.at[jt]], r2a_v, s2a).wait()
    pltpu.sync_copy(r1a_v, g1_hbm.at[pl.ds(base0 + jt * _CH, _CH)])
    pltpu.sync_copy(r2a_v, g2_hbm.at[pl.ds(base0 + jt * _CH, _CH)])


_NP = 10240
_TSL = _NP // _NS


def _sc_denom_body(logits_hbm, dst_hbm, gmax_hbm, ex_hbm, denom_hbm,
                   logit_v, ex_v, idx_v, gmax_v, zbuf_v, den_sh):
    sid = lax.axis_index("s")
    cid = lax.axis_index("c")
    wid = sid * _NC + cid
    z16 = jnp.zeros((16,), jnp.float32)
    for g in range(_TSL // 16):
        zbuf_v[pl.ds(g * 16, 16)] = z16
    pltpu.sync_copy(zbuf_v, den_sh.at[pl.ds(sid * _TSL, _TSL)])
    pltpu.sync_copy(logits_hbm.at[wid], logit_v)
    pltpu.sync_copy(dst_hbm.at[wid], idx_v)
    pltpu.sync_copy(gmax_hbm, gmax_v)
    gmax = gmax_v[...]
    plsc.subcore_barrier()

    def body(i, carry):
        for g in range(_CH // 16):
            lv = logit_v[i, pl.ds(g * 16, 16)]
            ex_v[i, pl.ds(g * 16, 16)] = jnp.exp(lv - gmax)
        pltpu.sync_copy(ex_v.at[i], den_sh.at[idx_v.at[i]], add=True)
        return carry

    lax.fori_loop(0, _NCHUNK, body, 0)
    pltpu.sync_copy(ex_v, ex_hbm.at[wid])
    plsc.subcore_barrier()
    pltpu.sync_copy(den_sh.at[pl.ds(sid * _TSL, _TSL)],
                    denom_hbm.at[cid, pl.ds(sid * _TSL, _TSL)])


def _sc_denom(logits3, dst3, gmax16):
    mesh = plsc.VectorSubcoreMesh(core_axis_name="c", subcore_axis_name="s")
    f = pl.kernel(
        _sc_denom_body,
        mesh=mesh,
        out_type=[
            jax.ShapeDtypeStruct((_NW, _NCHUNK, _CH), jnp.float32),
            jax.ShapeDtypeStruct((_NC, _NP), jnp.float32),
        ],
        scratch_types=[
            pltpu.VMEM((_NCHUNK, _CH), jnp.float32),
            pltpu.VMEM((_NCHUNK, _CH), jnp.float32),
            pltpu.VMEM((_NCHUNK, _CH), jnp.int32),
            pltpu.VMEM((16,), jnp.float32),
            pltpu.VMEM((_TSL,), jnp.float32),
            pltpu.VMEM_SHARED((_NP,), jnp.float32),
        ],
    )
    return f(logits3, dst3, gmax16)


_NP2 = _NP // 2
_TSL2 = _NP2 // _NS
_NCHUNK2 = _E // _NS // _CH


_NPQ = _NP // 2
_TSLQ = _NPQ // _NS


def _sc_msg_body(tn_hbm, src_hbm, dst_hbm, ex_hbm, denom_hbm, out_hbm,
                 idx_s_v, idx_d_v, idx_c_v, ex0_v, ex1_v, alpha_v,
                 d0_v, d1_v, rows0_v, rows1_v,
                 acc_sh, semr0, semr1, semd0, semd1):
    sid = lax.axis_index("s")
    cid = lax.axis_index("c")
    z16 = jnp.zeros((16,), jnp.float32)
    z16i = jnp.zeros((16,), jnp.int32)
    zf16 = jnp.zeros((16,), jnp.float32)
    rows = (rows0_v, rows1_v)
    dbuf = (d0_v, d1_v)
    ebuf = (ex0_v, ex1_v)
    semr = (semr0, semr1)
    semd = (semd0, semd1)
    pltpu.sync_copy(src_hbm.at[sid], idx_s_v)
    pltpu.sync_copy(dst_hbm.at[sid], idx_d_v)

    for p in range(1):
        for r in range(_CH):
            for g in range(8):
                rows0_v[r, pl.ds(g * 16, 16)] = z16
        for b in range(_TSLQ // _CH):
            pltpu.sync_copy(rows0_v, acc_sh.at[pl.ds(sid * _TSLQ + b * _CH, _CH)])
        plsc.subcore_barrier()
        lo = cid * _NP2 + p * _NPQ
        pltpu.async_copy(tn_hbm.at[idx_s_v.at[0]], rows0_v, semr0)
        pltpu.async_copy(denom_hbm.at[idx_d_v.at[0]], d0_v, semd0)
        pltpu.async_copy(ex_hbm.at[sid, 0], ex0_v, semd0)
        pltpu.async_copy(tn_hbm.at[idx_s_v.at[1]], rows1_v, semr1)
        pltpu.async_copy(denom_hbm.at[idx_d_v.at[1]], d1_v, semd1)
        pltpu.async_copy(ex_hbm.at[sid, 1], ex1_v, semd1)

        def pair(k, carry):
            for b in range(2):
                j = 2 * k + b
                pltpu.make_async_copy(
                    tn_hbm.at[idx_s_v.at[j]], rows[b], semr[b]).wait()
                pltpu.make_async_copy(
                    denom_hbm.at[idx_d_v.at[j]], dbuf[b], semd[b]).wait()
                pltpu.make_async_copy(
                    ex_hbm.at[sid, j], ebuf[b], semd[b]).wait()
                for g in range(_CH // 16):
                    d16 = dbuf[b][pl.ds(g * 16, 16)]
                    ex16 = ebuf[b][pl.ds(g * 16, 16)]
                    dv = idx_d_v[j, pl.ds(g * 16, 16)] - lo
                    inb = (dv >= 0) & (dv < _NPQ)
                    idx_c_v[pl.ds(g * 16, 16)] = jnp.where(inb, dv, z16i)
                    alpha_v[pl.ds(g * 16, 16)] = jnp.where(inb, ex16 / d16, zf16)
                for g16 in range(_CH // 16):
                    av16 = alpha_v[pl.ds(g16 * 16, 16)]
                    for jj in range(16):
                        r = g16 * 16 + jj
                        ar = av16[jj]
                        for g in range(8):
                            rows[b][r, pl.ds(g * 16, 16)] = (
                                rows[b][r, pl.ds(g * 16, 16)] * ar)
                pltpu.sync_copy(rows[b], acc_sh.at[idx_c_v], add=True)

                @pl.when(j + 2 < _NCHUNK2)
                def _():
                    pltpu.async_copy(
                        tn_hbm.at[idx_s_v.at[j + 2]], rows[b], semr[b])
                    pltpu.async_copy(
                        denom_hbm.at[idx_d_v.at[j + 2]], dbuf[b], semd[b])
                    pltpu.async_copy(ex_hbm.at[sid, j + 2], ebuf[b], semd[b])
            return carry

        lax.fori_loop(0, _NCHUNK2 // 2, pair, 0)
        plsc.subcore_barrier()
        pltpu.sync_copy(acc_sh.at[pl.ds(sid * _TSLQ, _TSLQ)],
                        out_hbm.at[cid + p, pl.ds(sid * _TSLQ, _TSLQ)])
        plsc.subcore_barrier()


def _sc_msg(table_n, src2, dst2, ex2, denom):
    h = table_n.shape[1]
    mesh = plsc.VectorSubcoreMesh(core_axis_name="c", subcore_axis_name="s")
    f = pl.kernel(
        _sc_msg_body,
        mesh=mesh,
        out_type=jax.ShapeDtypeStruct((2, _NPQ, h), jnp.float32),
        scratch_types=[
            pltpu.VMEM((_NCHUNK2, _CH), jnp.int32),
            pltpu.VMEM((_NCHUNK2, _CH), jnp.int32),
            pltpu.VMEM((_CH,), jnp.int32),
            pltpu.VMEM((_CH,), jnp.float32),
            pltpu.VMEM((_CH,), jnp.float32),
            pltpu.VMEM((_CH,), jnp.float32),
            pltpu.VMEM((_CH,), jnp.float32),
            pltpu.VMEM((_CH,), jnp.float32),
            pltpu.VMEM((_CH, h), jnp.float32),
            pltpu.VMEM((_CH, h), jnp.float32),
            pltpu.VMEM_SHARED((_NPQ, h), jnp.float32),
            pltpu.SemaphoreType.DMA,
            pltpu.SemaphoreType.DMA,
            pltpu.SemaphoreType.DMA,
            pltpu.SemaphoreType.DMA,
        ],
    )
    return f(table_n, src2, dst2, ex2, denom)


def _sc_gather2(table_i, table_j, dst, src):
    h = table_i.shape[1]
    dst3 = dst.reshape(_NW, _NCHUNK, _CH)
    src3 = src.reshape(_NW, _NCHUNK, _CH)
    mesh = plsc.VectorSubcoreMesh(core_axis_name="c", subcore_axis_name="s")
    f = pl.kernel(
        _sc_gather2_body,
        mesh=mesh,
        out_type=[
            jax.ShapeDtypeStruct((_E, h), jnp.float32),
            jax.ShapeDtypeStruct((_E, h), jnp.float32),
        ],
        scratch_types=[
            pltpu.VMEM((_NCHUNK, _CH), jnp.int32),
            pltpu.VMEM((_NCHUNK, _CH), jnp.int32),
            pltpu.VMEM((_CH, h), jnp.float32),
            pltpu.VMEM((_CH, h), jnp.float32),
            pltpu.VMEM((_CH, h), jnp.float32),
            pltpu.VMEM((_CH, h), jnp.float32),
            pltpu.SemaphoreType.DMA,
            pltpu.SemaphoreType.DMA,
            pltpu.SemaphoreType.DMA,
            pltpu.SemaphoreType.DMA,
        ],
    )
    return f(table_i, table_j, dst3, src3)


_PREC = jax.lax.Precision.DEFAULT


def _tc_proj_kernel(relu, x_ref, w_ref, b_ref, out_ref):
    x = x_ref[...]
    if relu:
        x = jnp.maximum(x, 0.0)
    out_ref[...] = (jnp.dot(x, w_ref[...], precision=_PREC,
                            preferred_element_type=jnp.float32) + b_ref[0])


def _tc_proj(x, w, b, relu):
    n, k = x.shape
    m = w.shape[1]
    blk = 2000
    return pl.pallas_call(
        functools.partial(_tc_proj_kernel, relu),
        grid=(n // blk,),
        in_specs=[
            pl.BlockSpec((blk, k), lambda i: (i, 0)),
            pl.BlockSpec((k, m), lambda i: (0, 0)),
            pl.BlockSpec(memory_space=pltpu.SMEM),
        ],
        out_specs=pl.BlockSpec((blk, m), lambda i: (i, 0)),
        out_shape=jax.ShapeDtypeStruct((n, m), jnp.float32),
    )(x, w, b)


def _tc_fused(g1, g2, c, we_c, av, we_next):
    e, h = g1.shape
    kc = c.shape[1]
    blk = 8000
    has_wec = we_c is not None
    has_next = we_next is not None
    av2 = av.reshape(h, 1)

    def kern(*refs):
        it = iter(refs)
        g1_ref = next(it)
        g2_ref = next(it)
        c_ref = next(it)
        we_ref = next(it) if has_wec else None
        av_ref = next(it)
        wn_ref = next(it) if has_next else None
        logit_ref = next(it)
        gmax_ref = next(it)
        cn_ref = next(it) if has_next else None
        i = pl.program_id(0)
        if has_wec:
            cterm = jnp.dot(c_ref[...], we_ref[...], precision=_PREC,
                            preferred_element_type=jnp.float32)
        else:
            cterm = c_ref[...]
        f = g1_ref[...] + g2_ref[...] + cterm
        e_act = jnp.where(f > 0, f, 0.2 * f)
        logits = jnp.dot(e_act, av_ref[...], precision=_PREC,
                         preferred_element_type=jnp.float32)
        logit_ref[...] = logits
        bmax = jnp.max(logits)

        @pl.when(i == 0)
        def _():
            gmax_ref[0, 0] = bmax

        @pl.when(i > 0)
        def _():
            gmax_ref[0, 0] = jnp.maximum(gmax_ref[0, 0], bmax)

        if has_next:
            cn_ref[...] = jnp.dot(f, wn_ref[...], precision=_PREC,
                                  preferred_element_type=jnp.float32)

    in_specs = [
        pl.BlockSpec((blk, h), lambda i: (i, 0)),
        pl.BlockSpec((blk, h), lambda i: (i, 0)),
        pl.BlockSpec((blk, kc), lambda i: (i, 0)),
    ]
    args = [g1, g2, c]
    if has_wec:
        in_specs.append(pl.BlockSpec((kc, h), lambda i: (0, 0)))
        args.append(we_c)
    in_specs.append(pl.BlockSpec((h, 1), lambda i: (0, 0)))
    args.append(av2)
    out_specs = [
        pl.BlockSpec((blk, 1), lambda i: (i, 0)),
        pl.BlockSpec((1, 1), lambda i: (0, 0), memory_space=pltpu.SMEM),
    ]
    out_shape = [
        jax.ShapeDtypeStruct((e, 1), jnp.float32),
        jax.ShapeDtypeStruct((1, 1), jnp.float32),
    ]
    if has_next:
        in_specs.append(pl.BlockSpec((h, h), lambda i: (0, 0)))
        args.append(we_next)
        out_specs.append(pl.BlockSpec((blk, h), lambda i: (i, 0)))
        out_shape.append(jax.ShapeDtypeStruct((e, h), jnp.float32))
    res = pl.pallas_call(
        kern, grid=(e // blk,), in_specs=in_specs,
        out_specs=out_specs, out_shape=out_shape,
    )(*args)
    c_next = res[2] if has_next else None
    return res[0].reshape(e), res[1][0, 0], c_next


def _layer(x, src, dst, c, we_c, Wn, Wi, Wj, av, n, We_next, relu_in):
    wcat = jnp.concatenate([Wi, Wj, Wn], axis=1)
    zb = jnp.zeros((1,), jnp.float32)
    tbl = _tc_proj(x, wcat, zb, relu_in)
    h = Wn.shape[1]
    xWi = tbl[:, :h]
    xWj = tbl[:, h:2 * h]
    xWn = tbl[:, 2 * h:]
    g1, g2 = _sc_gather2(xWi, xWj, dst, src)
    logits, gmax, c_next = _tc_fused(g1, g2, c, we_c, av, We_next)
    logits3 = logits.reshape(_NW, _NCHUNK, _CH)
    dst3 = dst.reshape(_NW, _NCHUNK, _CH)
    gmax16 = jnp.full((16,), gmax, jnp.float32)
    ex3, denom2 = _sc_denom(logits3, dst3, gmax16)
    denom = denom2[0] + denom2[1] + 1e-16
    src2 = src.reshape(_NS, _NCHUNK2, _CH)
    dst2 = dst.reshape(_NS, _NCHUNK2, _CH)
    ex2 = ex3.reshape(_NS, _NCHUNK2, _CH)
    out4 = _sc_msg(xWn, src2, dst2, ex2, denom)
    out = out4.reshape(2 * _NPQ, -1)[:n]
    return out, c_next


def kernel(x, edge_index, edge_attr, Wn1, Wi1, Wj1, We1, av1, Wn2, Wi2, Wj2, We2, av2, Wn3, Wi3, Wj3, We3, av3, Wc, bc):
    n = x.shape[0]
    src = edge_index[0]
    dst = edge_index[1]
    h, c2 = _layer(x, src, dst, edge_attr, We1, Wn1, Wi1, Wj1, av1, n, We2,
                   relu_in=False)
    h, c3 = _layer(h, src, dst, c2, None, Wn2, Wi2, Wj2, av2, n, We3,
                   relu_in=True)
    h, _ = _layer(h, src, dst, c3, None, Wn3, Wi3, Wj3, av3, n, None,
                  relu_in=True)
    return _tc_proj(h, Wc, bc, relu=True)


# speedup vs baseline: 2.0293x; 1.0157x over previous
import functools

import jax
import jax.numpy as jnp
from jax import lax
from jax.experimental import pallas as pl
from jax.experimental.pallas import tpu as pltpu
from jax.experimental.pallas import tpu_sc as plsc

_NC, _NS = 2, 16
_NW = _NC * _NS
_E = 320000
_BPW = _E // _NW
_CH = 80
_NCHUNK = _BPW // _CH


def _sc_gather2_body(ti_hbm, tj_hbm, dst_hbm, src_hbm, g1_hbm, g2_hbm,
                     idx_d_v, idx_s_v, r1a_v, r1b_v, r2a_v, r2b_v,
                     s1a, s1b, s2a, s2b):
    wid = lax.axis_index("s") * _NC + lax.axis_index("c")
    base0 = wid * _BPW
    pltpu.sync_copy(dst_hbm.at[wid], idx_d_v)
    pltpu.sync_copy(src_hbm.at[wid], idx_s_v)
    r1 = (r1a_v, r1b_v)
    r2 = (r2a_v, r2b_v)
    s1 = (s1a, s1b)
    s2 = (s2a, s2b)
    pltpu.async_copy(ti_hbm.at[idx_d_v.at[0]], r1a_v, s1a)
    pltpu.async_copy(tj_hbm.at[idx_s_v.at[0]], r2a_v, s2a)
    pltpu.async_copy(ti_hbm.at[idx_d_v.at[1]], r1b_v, s1b)
    pltpu.async_copy(tj_hbm.at[idx_s_v.at[1]], r2b_v, s2b)

    def pair(k, carry):
        for b in range(2):
            j = 2 * k + b
            base = base0 + j * _CH
            pltpu.make_async_copy(ti_hbm.at[idx_d_v.at[j]], r1[b], s1[b]).wait()
            pltpu.make_async_copy(tj_hbm.at[idx_s_v.at[j]], r2[b], s2[b]).wait()
            pltpu.sync_copy(r1[b], g1_hbm.at[pl.ds(base, _CH)])
            pltpu.sync_copy(r2[b], g2_hbm.at[pl.ds(base, _CH)])

            @pl.when(j + 2 < _NCHUNK)
            def _():
                pltpu.async_copy(ti_hbm.at[idx_d_v.at[j + 2]], r1[b], s1[b])
                pltpu.async_copy(tj_hbm.at[idx_s_v.at[j + 2]], r2[b], s2[b])
        return carry

    lax.fori_loop(0, _NCHUNK // 2, pair, 0)
    jt = _NCHUNK - 1
    pltpu.make_async_copy(ti_hbm.at[idx_d_v.at[jt]], r1a_v, s1a).wait()
    pltpu.make_async_copy(tj_hbm.at[idx_s_v.at[jt]], r2a_v, s2a).wait()
    pltpu.sync_copy(r1a_v, g1_hbm.at[pl.ds(base0 + jt * _CH, _CH)])
    pltpu.sync_copy(r2a_v, g2_hbm.at[pl.ds(base0 + jt * _CH, _CH)])


_NP = 10240
_TSL = _NP // _NS


def _sc_denom_body(logits_hbm, dst_hbm, gmax_hbm, ex_hbm, denom_hbm,
                   logit_v, ex_v, idx_v, gmax_v, zbuf_v, den_sh):
    sid = lax.axis_index("s")
    cid = lax.axis_index("c")
    wid = sid * _NC + cid
    z16 = jnp.zeros((16,), jnp.float32)
    for g in range(_TSL // 16):
        zbuf_v[pl.ds(g * 16, 16)] = z16
    pltpu.sync_copy(zbuf_v, den_sh.at[pl.ds(sid * _TSL, _TSL)])
    pltpu.sync_copy(logits_hbm.at[wid], logit_v)
    pltpu.sync_copy(dst_hbm.at[wid], idx_v)
    pltpu.sync_copy(gmax_hbm, gmax_v)
    gmax = gmax_v[...]
    plsc.subcore_barrier()

    def body(i, carry):
        for g in range(_CH // 16):
            lv = logit_v[i, pl.ds(g * 16, 16)]
            ex_v[i, pl.ds(g * 16, 16)] = jnp.exp(lv - gmax)
        pltpu.sync_copy(ex_v.at[i], den_sh.at[idx_v.at[i]], add=True)
        return carry

    lax.fori_loop(0, _NCHUNK, body, 0)
    pltpu.sync_copy(ex_v, ex_hbm.at[wid])
    plsc.subcore_barrier()
    pltpu.sync_copy(den_sh.at[pl.ds(sid * _TSL, _TSL)],
                    denom_hbm.at[cid, pl.ds(sid * _TSL, _TSL)])


def _sc_denom(logits3, dst3, gmax16):
    mesh = plsc.VectorSubcoreMesh(core_axis_name="c", subcore_axis_name="s")
    f = pl.kernel(
        _sc_denom_body,
        mesh=mesh,
        out_type=[
            jax.ShapeDtypeStruct((_NW, _NCHUNK, _CH), jnp.float32),
            jax.ShapeDtypeStruct((_NC, _NP), jnp.float32),
        ],
        scratch_types=[
            pltpu.VMEM((_NCHUNK, _CH), jnp.float32),
            pltpu.VMEM((_NCHUNK, _CH), jnp.float32),
            pltpu.VMEM((_NCHUNK, _CH), jnp.int32),
            pltpu.VMEM((16,), jnp.float32),
            pltpu.VMEM((_TSL,), jnp.float32),
            pltpu.VMEM_SHARED((_NP,), jnp.float32),
        ],
    )
    return f(logits3, dst3, gmax16)


_NP2 = _NP // 2
_TSL2 = _NP2 // _NS
_NCHUNK2 = _E // _NS // _CH


_NPQ = _NP // 2
_TSLQ = _NPQ // _NS


def _sc_msg_body(tn_hbm, src_hbm, dst_hbm, ex_hbm, denom_hbm, out_hbm,
                 idx_s_v, idx_d_v, idx_c_v, ex0_v, ex1_v, alpha_v,
                 d0_v, d1_v, rows0_v, rows1_v,
                 acc_sh, semr0, semr1, semd0, semd1):
    sid = lax.axis_index("s")
    cid = lax.axis_index("c")
    z16 = jnp.zeros((16,), jnp.float32)
    t16i = jnp.full((16,), _NPQ, jnp.int32)
    zf16 = jnp.zeros((16,), jnp.float32)
    rows = (rows0_v, rows1_v)
    dbuf = (d0_v, d1_v)
    ebuf = (ex0_v, ex1_v)
    semr = (semr0, semr1)
    semd = (semd0, semd1)
    pltpu.sync_copy(src_hbm.at[sid], idx_s_v)
    pltpu.sync_copy(dst_hbm.at[sid], idx_d_v)

    for p in range(1):
        for r in range(_CH):
            for g in range(8):
                rows0_v[r, pl.ds(g * 16, 16)] = z16
        for b in range(_TSLQ // _CH):
            pltpu.sync_copy(rows0_v, acc_sh.at[pl.ds(sid * _TSLQ + b * _CH, _CH)])

        @pl.when(sid == 0)
        def _():
            pltpu.sync_copy(rows0_v.at[pl.ds(0, 8)], acc_sh.at[pl.ds(_NPQ, 8)])
        plsc.subcore_barrier()
        lo = cid * _NP2 + p * _NPQ
        pltpu.async_copy(tn_hbm.at[idx_s_v.at[0]], rows0_v, semr0)
        pltpu.async_copy(denom_hbm.at[idx_d_v.at[0]], d0_v, semd0)
        pltpu.async_copy(ex_hbm.at[sid, 0], ex0_v, semd0)
        pltpu.async_copy(tn_hbm.at[idx_s_v.at[1]], rows1_v, semr1)
        pltpu.async_copy(denom_hbm.at[idx_d_v.at[1]], d1_v, semd1)
        pltpu.async_copy(ex_hbm.at[sid, 1], ex1_v, semd1)

        def pair(k, carry):
            for b in range(2):
                j = 2 * k + b
                pltpu.make_async_copy(
                    tn_hbm.at[idx_s_v.at[j]], rows[b], semr[b]).wait()
                pltpu.make_async_copy(
                    denom_hbm.at[idx_d_v.at[j]], dbuf[b], semd[b]).wait()
                pltpu.make_async_copy(
                    ex_hbm.at[sid, j], ebuf[b], semd[b]).wait()
                for g in range(_CH // 16):
                    d16 = dbuf[b][pl.ds(g * 16, 16)]
                    ex16 = ebuf[b][pl.ds(g * 16, 16)]
                    dv = idx_d_v[j, pl.ds(g * 16, 16)] - lo
                    inb = (dv >= 0) & (dv < _NPQ)
                    idx_c_v[pl.ds(g * 16, 16)] = jnp.where(inb, dv, t16i)
                    alpha_v[pl.ds(g * 16, 16)] = jnp.where(inb, ex16 / d16, zf16)
                for g16 in range(_CH // 16):
                    av16 = alpha_v[pl.ds(g16 * 16, 16)]
                    for jj in range(16):
                        r = g16 * 16 + jj
                        ar = av16[jj]

                        @pl.when(ar > 0.0)
                        def _():
                            for g in range(8):
                                rows[b][r, pl.ds(g * 16, 16)] = (
                                    rows[b][r, pl.ds(g * 16, 16)] * ar)
                pltpu.sync_copy(rows[b], acc_sh.at[idx_c_v], add=True)

                @pl.when(j + 2 < _NCHUNK2)
                def _():
                    pltpu.async_copy(
                        tn_hbm.at[idx_s_v.at[j + 2]], rows[b], semr[b])
                    pltpu.async_copy(
                        denom_hbm.at[idx_d_v.at[j + 2]], dbuf[b], semd[b])
                    pltpu.async_copy(ex_hbm.at[sid, j + 2], ebuf[b], semd[b])
            return carry

        lax.fori_loop(0, _NCHUNK2 // 2, pair, 0)
        plsc.subcore_barrier()
        pltpu.sync_copy(acc_sh.at[pl.ds(sid * _TSLQ, _TSLQ)],
                        out_hbm.at[cid + p, pl.ds(sid * _TSLQ, _TSLQ)])
        plsc.subcore_barrier()


def _sc_msg(table_n, src2, dst2, ex2, denom):
    h = table_n.shape[1]
    mesh = plsc.VectorSubcoreMesh(core_axis_name="c", subcore_axis_name="s")
    f = pl.kernel(
        _sc_msg_body,
        mesh=mesh,
        out_type=jax.ShapeDtypeStruct((2, _NPQ, h), jnp.float32),
        scratch_types=[
            pltpu.VMEM((_NCHUNK2, _CH), jnp.int32),
            pltpu.VMEM((_NCHUNK2, _CH), jnp.int32),
            pltpu.VMEM((_CH,), jnp.int32),
            pltpu.VMEM((_CH,), jnp.float32),
            pltpu.VMEM((_CH,), jnp.float32),
            pltpu.VMEM((_CH,), jnp.float32),
            pltpu.VMEM((_CH,), jnp.float32),
            pltpu.VMEM((_CH,), jnp.float32),
            pltpu.VMEM((_CH, h), jnp.float32),
            pltpu.VMEM((_CH, h), jnp.float32),
            pltpu.VMEM_SHARED((_NPQ + 8, h), jnp.float32),
            pltpu.SemaphoreType.DMA,
            pltpu.SemaphoreType.DMA,
            pltpu.SemaphoreType.DMA,
            pltpu.SemaphoreType.DMA,
        ],
    )
    return f(table_n, src2, dst2, ex2, denom)


def _sc_gather2(table_i, table_j, dst, src):
    h = table_i.shape[1]
    dst3 = dst.reshape(_NW, _NCHUNK, _CH)
    src3 = src.reshape(_NW, _NCHUNK, _CH)
    mesh = plsc.VectorSubcoreMesh(core_axis_name="c", subcore_axis_name="s")
    f = pl.kernel(
        _sc_gather2_body,
        mesh=mesh,
        out_type=[
            jax.ShapeDtypeStruct((_E, h), jnp.float32),
            jax.ShapeDtypeStruct((_E, h), jnp.float32),
        ],
        scratch_types=[
            pltpu.VMEM((_NCHUNK, _CH), jnp.int32),
            pltpu.VMEM((_NCHUNK, _CH), jnp.int32),
            pltpu.VMEM((_CH, h), jnp.float32),
            pltpu.VMEM((_CH, h), jnp.float32),
            pltpu.VMEM((_CH, h), jnp.float32),
            pltpu.VMEM((_CH, h), jnp.float32),
            pltpu.SemaphoreType.DMA,
            pltpu.SemaphoreType.DMA,
            pltpu.SemaphoreType.DMA,
            pltpu.SemaphoreType.DMA,
        ],
    )
    return f(table_i, table_j, dst3, src3)


_PREC = jax.lax.Precision.DEFAULT


def _tc_proj_kernel(relu, x_ref, w_ref, b_ref, out_ref):
    x = x_ref[...]
    if relu:
        x = jnp.maximum(x, 0.0)
    out_ref[...] = (jnp.dot(x, w_ref[...], precision=_PREC,
                            preferred_element_type=jnp.float32) + b_ref[0])


def _tc_proj(x, w, b, relu):
    n, k = x.shape
    m = w.shape[1]
    blk = 2000
    return pl.pallas_call(
        functools.partial(_tc_proj_kernel, relu),
        grid=(n // blk,),
        in_specs=[
            pl.BlockSpec((blk, k), lambda i: (i, 0)),
            pl.BlockSpec((k, m), lambda i: (0, 0)),
            pl.BlockSpec(memory_space=pltpu.SMEM),
        ],
        out_specs=pl.BlockSpec((blk, m), lambda i: (i, 0)),
        out_shape=jax.ShapeDtypeStruct((n, m), jnp.float32),
    )(x, w, b)


def _tc_fused(g1, g2, c, we_c, av, we_next):
    e, h = g1.shape
    kc = c.shape[1]
    blk = 8000
    has_wec = we_c is not None
    has_next = we_next is not None
    av2 = av.reshape(h, 1)

    def kern(*refs):
        it = iter(refs)
        g1_ref = next(it)
        g2_ref = next(it)
        c_ref = next(it)
        we_ref = next(it) if has_wec else None
        av_ref = next(it)
        wn_ref = next(it) if has_next else None
        logit_ref = next(it)
        gmax_ref = next(it)
        cn_ref = next(it) if has_next else None
        i = pl.program_id(0)
        if has_wec:
            cterm = jnp.dot(c_ref[...], we_ref[...], precision=_PREC,
                            preferred_element_type=jnp.float32)
        else:
            cterm = c_ref[...]
        f = g1_ref[...] + g2_ref[...] + cterm
        e_act = jnp.where(f > 0, f, 0.2 * f)
        logits = jnp.dot(e_act, av_ref[...], precision=_PREC,
                         preferred_element_type=jnp.float32)
        logit_ref[...] = logits
        bmax = jnp.max(logits)

        @pl.when(i == 0)
        def _():
            gmax_ref[0, 0] = bmax

        @pl.when(i > 0)
        def _():
            gmax_ref[0, 0] = jnp.maximum(gmax_ref[0, 0], bmax)

        if has_next:
            cn_ref[...] = jnp.dot(f, wn_ref[...], precision=_PREC,
                                  preferred_element_type=jnp.float32)

    in_specs = [
        pl.BlockSpec((blk, h), lambda i: (i, 0)),
        pl.BlockSpec((blk, h), lambda i: (i, 0)),
        pl.BlockSpec((blk, kc), lambda i: (i, 0)),
    ]
    args = [g1, g2, c]
    if has_wec:
        in_specs.append(pl.BlockSpec((kc, h), lambda i: (0, 0)))
        args.append(we_c)
    in_specs.append(pl.BlockSpec((h, 1), lambda i: (0, 0)))
    args.append(av2)
    out_specs = [
        pl.BlockSpec((blk, 1), lambda i: (i, 0)),
        pl.BlockSpec((1, 1), lambda i: (0, 0), memory_space=pltpu.SMEM),
    ]
    out_shape = [
        jax.ShapeDtypeStruct((e, 1), jnp.float32),
        jax.ShapeDtypeStruct((1, 1), jnp.float32),
    ]
    if has_next:
        in_specs.append(pl.BlockSpec((h, h), lambda i: (0, 0)))
        args.append(we_next)
        out_specs.append(pl.BlockSpec((blk, h), lambda i: (i, 0)))
        out_shape.append(jax.ShapeDtypeStruct((e, h), jnp.float32))
    res = pl.pallas_call(
        kern, grid=(e // blk,), in_specs=in_specs,
        out_specs=out_specs, out_shape=out_shape,
    )(*args)
    c_next = res[2] if has_next else None
    return res[0].reshape(e), res[1][0, 0], c_next


def _layer(x, src, dst, c, we_c, Wn, Wi, Wj, av, n, We_next, relu_in):
    wcat = jnp.concatenate([Wi, Wj, Wn], axis=1)
    zb = jnp.zeros((1,), jnp.float32)
    tbl = _tc_proj(x, wcat, zb, relu_in)
    h = Wn.shape[1]
    xWi = tbl[:, :h]
    xWj = tbl[:, h:2 * h]
    xWn = tbl[:, 2 * h:]
    g1, g2 = _sc_gather2(xWi, xWj, dst, src)
    logits, gmax, c_next = _tc_fused(g1, g2, c, we_c, av, We_next)
    logits3 = logits.reshape(_NW, _NCHUNK, _CH)
    dst3 = dst.reshape(_NW, _NCHUNK, _CH)
    gmax16 = jnp.full((16,), gmax, jnp.float32)
    ex3, denom2 = _sc_denom(logits3, dst3, gmax16)
    denom = denom2[0] + denom2[1] + 1e-16
    src2 = src.reshape(_NS, _NCHUNK2, _CH)
    dst2 = dst.reshape(_NS, _NCHUNK2, _CH)
    ex2 = ex3.reshape(_NS, _NCHUNK2, _CH)
    out4 = _sc_msg(xWn, src2, dst2, ex2, denom)
    out = out4.reshape(2 * _NPQ, -1)[:n]
    return out, c_next


def kernel(x, edge_index, edge_attr, Wn1, Wi1, Wj1, We1, av1, Wn2, Wi2, Wj2, We2, av2, Wn3, Wi3, Wj3, We3, av3, Wc, bc):
    n = x.shape[0]
    src = edge_index[0]
    dst = edge_index[1]
    h, c2 = _layer(x, src, dst, edge_attr, We1, Wn1, Wi1, Wj1, av1, n, We2,
                   relu_in=False)
    h, c3 = _layer(h, src, dst, c2, None, Wn2, Wi2, Wj2, av2, n, We3,
                   relu_in=True)
    h, _ = _layer(h, src, dst, c3, None, Wn3, Wi3, Wj3, av3, n, None,
                  relu_in=True)
    return _tc_proj(h, Wc, bc, relu=True)
